# Initial kernel scaffold; baseline (speedup 1.0000x reference)
import jax
import jax.numpy as jnp
from jax.experimental import pallas as pl


def kernel(x, edge_index, liquor_idx, ingredient_idx, W1, a_src1, a_dst1, b1, W2, a_src2, a_dst2, b2, W3, a_src3, a_dst3, b3, Wm1, bm1, Wm2, bm2, Wm3, bm3):
    raise NotImplementedError("write your pallas kernel here")


# speedup vs baseline: 6.9474x; 6.9474x over previous
import functools

import jax
import jax.numpy as jnp
from jax import lax
from jax.experimental import pallas as pl
from jax.experimental.pallas import tpu as pltpu
from jax.experimental.pallas import tpu_sc as plsc

F32 = jnp.float32
I32 = jnp.int32

N = 10000
NPAD = 10240
D = 128
HID = 64
HEADS = 8
F = 512
E = 330000
B = 128
B1 = 32
NC, NS = 2, 16
NW = NC * NS
NB2 = 82
Q1 = NB2 * B
NB1 = Q1 // B1
EPAD = NW * Q1
NSC = 10112
RPT = NSC // NS

_mesh = plsc.VectorSubcoreMesh(
    core_axis_name="c", subcore_axis_name="s", num_cores=NC, num_subcores=NS)


def _p1_body(nex, *args):
  (as_hbm, ad_hbm, src_hbm, dst_hbm, z128_hbm) = args[:5]
  ex_outs = args[5:5 + nex]
  den_hbm = args[5 + nex]
  scr = args[6 + nex:]
  den_sh, sidx_v, didx_v, as_v, ad_v, exd_v = scr[:6]
  exc_vs = scr[6:6 + nex]
  sem = scr[6 + nex]

  cid = lax.axis_index("c")
  tid = lax.axis_index("s")
  wid = tid * NC + cid
  pltpu.sync_copy(z128_hbm.at[pl.ds(0, RPT)], den_sh.at[pl.ds(tid * RPT, RPT)])
  plsc.subcore_barrier()
  base = wid * Q1

  def batch(bi, _):
    e0 = base + bi * B1
    pltpu.sync_copy(src_hbm.at[pl.ds(e0, B1)], sidx_v)
    pltpu.sync_copy(dst_hbm.at[pl.ds(e0, B1)], didx_v)
    pltpu.async_copy(as_hbm.at[sidx_v], as_v, sem).wait()
    pltpu.async_copy(ad_hbm.at[didx_v], ad_v, sem).wait()

    def edge(i, _):
      for g in range(2 * nex):
        e16 = as_v[i, pl.ds(16 * g, 16)] + ad_v[i, pl.ds(16 * g, 16)]
        e16 = jnp.where(e16 >= 0.0, e16, e16 * 0.2)
        ex = jnp.exp(e16)
        exd_v[i, pl.ds(16 * g, 16)] = ex
        exc_vs[g // 2][i, pl.ds(16 * (g % 2), 16)] = ex
      return 0

    lax.fori_loop(0, B1, edge, 0, unroll=2)
    for c in range(nex):
      pltpu.sync_copy(exc_vs[c], ex_outs[c].at[pl.ds(e0, B1)])
    pltpu.sync_copy(exd_v, den_sh.at[didx_v], add=True)
    return 0

  lax.fori_loop(0, NB1, batch, 0)
  plsc.subcore_barrier()
  pltpu.sync_copy(den_sh.at[pl.ds(tid * RPT, RPT)],
                  den_hbm.at[cid, pl.ds(tid * RPT, RPT)])


def _phase1(as_tab, ad_tab, src, dst, z128, nex):
  fn = pl.kernel(
      functools.partial(_p1_body, nex),
      out_type=(tuple(jax.ShapeDtypeStruct((EPAD, 32), F32)
                      for _ in range(nex))
                + (jax.ShapeDtypeStruct((NC, NPAD, 128), F32),)),
      mesh=_mesh,
      scratch_types=[
          pltpu.VMEM_SHARED((NSC, 128), F32),
          pltpu.VMEM((B1,), I32),
          pltpu.VMEM((B1,), I32),
          pltpu.VMEM((B1, 128), F32),
          pltpu.VMEM((B1, 128), F32),
          pltpu.VMEM((B1, 128), F32),
      ] + [pltpu.VMEM((B1, 32), F32) for _ in range(nex)]
      + [pltpu.SemaphoreType.DMA],
  )
  outs = fn(as_tab, ad_tab, src, dst, z128)
  return outs[:nex], outs[nex]


def _p2_body(h_hbm, exc_hbm, src_hbm, dst_hbm, z128_hbm,
             raw_hbm,
             acc_sh, sidx_v, didx_v, ex_v, g_v, sem):
  cid = lax.axis_index("c")
  tid = lax.axis_index("s")
  wid = tid * NC + cid
  pltpu.sync_copy(z128_hbm.at[pl.ds(0, RPT)], acc_sh.at[pl.ds(tid * RPT, RPT)])
  plsc.subcore_barrier()
  base = wid * Q1

  def batch(bi, _):
    e0 = base + bi * B
    pltpu.sync_copy(src_hbm.at[pl.ds(e0, B)], sidx_v)
    pltpu.sync_copy(dst_hbm.at[pl.ds(e0, B)], didx_v)
    pltpu.sync_copy(exc_hbm.at[pl.ds(e0, B)], ex_v)
    pltpu.async_copy(h_hbm.at[sidx_v], g_v, sem).wait()

    def edge(i, _):
      s0 = ex_v[i, pl.ds(0, 16)]
      s1 = ex_v[i, pl.ds(16, 16)]
      for j in range(8):
        s = s0 if j < 4 else s1
        g_v[i, pl.ds(j * 16, 16)] = g_v[i, pl.ds(j * 16, 16)] * s
      return 0

    lax.fori_loop(0, B, edge, 0, unroll=2)
    pltpu.sync_copy(g_v, acc_sh.at[didx_v], add=True)
    return 0

  lax.fori_loop(0, NB2, batch, 0)
  plsc.subcore_barrier()
  pltpu.sync_copy(acc_sh.at[pl.ds(tid * RPT, RPT)],
                  raw_hbm.at[cid, pl.ds(tid * RPT, RPT)])


def _phase2_chunk(h_chunk, exc, src, dst, z128):
  fn = pl.kernel(
      _p2_body,
      out_type=jax.ShapeDtypeStruct((NC, NPAD, 128), F32),
      mesh=_mesh,
      scratch_types=[
          pltpu.VMEM_SHARED((NSC, 128), F32),
          pltpu.VMEM((B,), I32),
          pltpu.VMEM((B,), I32),
          pltpu.VMEM((B, 32), F32),
          pltpu.VMEM((B, 128), F32),
          pltpu.SemaphoreType.DMA,
      ],
  )
  return fn(h_chunk, exc, src, dst, z128)


def _phase2(h_chunks, exs, src, dst, z128):
  return [_phase2_chunk(h_chunks[c], exs[c], src, dst, z128)
          for c in range(4)]


def _pair_body(h3_hbm, liq_hbm, ing_hbm, l_out, i_out, idx_v, rows_v, sem):
  cid = lax.axis_index("c")
  tid = lax.axis_index("s")
  wid = tid * NC + cid
  base = wid * 32
  pltpu.sync_copy(liq_hbm.at[pl.ds(base, 32)], idx_v)
  pltpu.async_copy(h3_hbm.at[idx_v], rows_v, sem).wait()
  pltpu.sync_copy(rows_v, l_out.at[pl.ds(base, 32)])
  pltpu.sync_copy(ing_hbm.at[pl.ds(base, 32)], idx_v)
  pltpu.async_copy(h3_hbm.at[idx_v], rows_v, sem).wait()
  pltpu.sync_copy(rows_v, i_out.at[pl.ds(base, 32)])


def _pair_gather(h3, liq, ing):
  fn = pl.kernel(
      _pair_body,
      out_type=(jax.ShapeDtypeStruct((1024, 128), F32),
                jax.ShapeDtypeStruct((1024, 128), F32)),
      mesh=_mesh,
      scratch_types=[
          pltpu.VMEM((32,), I32),
          pltpu.VMEM((32, 128), F32),
          pltpu.SemaphoreType.DMA,
      ],
  )
  return fn(h3, liq, ing)


_BLK = 256
_GRID = NPAD // _BLK


def _lift1_body(x_ref, w_ref, ws_ref, wd_ref, h0, h1, h2, h3, as_ref, ad_ref):
  h = jnp.dot(x_ref[...], w_ref[...], preferred_element_type=F32)
  h0[...] = h[:, 0:128]
  h1[...] = h[:, 128:256]
  h2[...] = h[:, 256:384]
  h3[...] = h[:, 384:512]
  as_ref[...] = jnp.dot(h, ws_ref[...], preferred_element_type=F32)
  ad_ref[...] = jnp.dot(h, wd_ref[...], preferred_element_type=F32)


def _lift1(xp, W1, ws, wd):
  return pl.pallas_call(
      _lift1_body,
      grid=(_GRID,),
      in_specs=[
          pl.BlockSpec((_BLK, D), lambda i: (i, 0)),
          pl.BlockSpec((D, F), lambda i: (0, 0)),
          pl.BlockSpec((F, 128), lambda i: (0, 0)),
          pl.BlockSpec((F, 128), lambda i: (0, 0)),
      ],
      out_specs=[pl.BlockSpec((_BLK, 128), lambda i: (i, 0))] * 6,
      out_shape=[jax.ShapeDtypeStruct((NPAD, 128), F32)] * 6,
  )(xp, W1, ws, wd)


def _comb_act(r_refs, den_ref, b_ref):
  d = den_ref[0] + den_ref[1]
  inv = 1.0 / (d + 1e-16)
  cols = []
  for h in range(HEADS):
    rr = r_refs[h // 2]
    blk = (rr[0] + rr[1])[:, (h % 2) * 64:(h % 2) * 64 + 64]
    cols.append(blk * inv[:, 16 * h:16 * h + 1])
  act = jnp.concatenate(cols, axis=1) + b_ref[...]
  return jnp.maximum(act, 0.0)


def _comb_lift_body(cw, r0, r1, r2, r3, den_ref, b_ref, w_ref,
                    ws_ref, wd_ref, *outs):
  act = _comb_act([r0, r1, r2, r3], den_ref, b_ref)
  h = jnp.dot(act, w_ref[...], preferred_element_type=F32)
  nchunks = cw * 0 + (F // 128 if cw == 128 else 1)
  hc = h
  if cw < 128:
    hc = jnp.concatenate([h, jnp.zeros((h.shape[0], 128 - cw), F32)], axis=1)
  for c in range(nchunks):
    outs[c][...] = hc[:, c * 128:c * 128 + 128] if cw == 128 else hc
  as_out = jnp.dot(h, ws_ref[...], preferred_element_type=F32)
  ad_out = jnp.dot(h, wd_ref[...], preferred_element_type=F32)
  outs[nchunks][...] = as_out
  outs[nchunks + 1][...] = ad_out


def _comb_lift(raws, den, b, W, ws, wd, fout):
  nchunks = 4 if fout == F else 1
  cw = fout // nchunks
  body = functools.partial(_comb_lift_body, cw)
  return pl.pallas_call(
      body,
      grid=(_GRID,),
      in_specs=[pl.BlockSpec((NC, _BLK, 128), lambda i: (0, i, 0))] * 4
      + [
          pl.BlockSpec((NC, _BLK, 128), lambda i: (0, i, 0)),
          pl.BlockSpec((1, F), lambda i: (0, 0)),
          pl.BlockSpec((F, fout), lambda i: (0, 0)),
          pl.BlockSpec((fout, 128), lambda i: (0, 0)),
          pl.BlockSpec((fout, 128), lambda i: (0, 0)),
      ],
      out_specs=[pl.BlockSpec((_BLK, 128), lambda i: (i, 0))] * (nchunks + 2),
      out_shape=[jax.ShapeDtypeStruct((NPAD, 128), F32)] * (nchunks + 2),
  )(*raws, den, b, W, ws, wd)


def _comb3_body(r_ref, den_ref, b_ref, out_ref):
  d = den_ref[0] + den_ref[1]
  inv = 1.0 / (d[:, 0:1] + 1e-16)
  out_ref[...] = (r_ref[0] + r_ref[1]) * inv + b_ref[...]


def _comb3(raw3, den3, b3p):
  return pl.pallas_call(
      _comb3_body,
      grid=(_GRID,),
      in_specs=[
          pl.BlockSpec((NC, _BLK, 128), lambda i: (0, i, 0)),
          pl.BlockSpec((NC, _BLK, 128), lambda i: (0, i, 0)),
          pl.BlockSpec((1, 128), lambda i: (0, 0)),
      ],
      out_specs=pl.BlockSpec((_BLK, 128), lambda i: (i, 0)),
      out_shape=jax.ShapeDtypeStruct((NPAD, 128), F32),
  )(raw3, den3, b3p)


def _mlp_body(l_ref, i_ref, w1a_ref, w1b_ref, b1_ref, w2_ref, b2_ref,
              w3_ref, b3_ref, out_ref):
  z = (jnp.dot(l_ref[...], w1a_ref[...], preferred_element_type=F32)
       + jnp.dot(i_ref[...], w1b_ref[...], preferred_element_type=F32)
       + b1_ref[...])
  z = jnp.maximum(z, 0.0)
  z = jnp.dot(z, w2_ref[...], preferred_element_type=F32) + b2_ref[...]
  z = jnp.maximum(z, 0.0)
  z = jnp.dot(z, w3_ref[...], preferred_element_type=F32) + b3_ref[...]
  out_ref[...] = 1.0 / (1.0 + jnp.exp(-z))


def _mlp(l_rows, i_rows, w1a, w1b, bm1, w2p, bm2p, w3p, bm3p):
  return pl.pallas_call(
      _mlp_body,
      grid=(1,),
      in_specs=[
          pl.BlockSpec((1024, 128), lambda i: (0, 0)),
          pl.BlockSpec((1024, 128), lambda i: (0, 0)),
          pl.BlockSpec((128, 64), lambda i: (0, 0)),
          pl.BlockSpec((128, 64), lambda i: (0, 0)),
          pl.BlockSpec((1, 64), lambda i: (0, 0)),
          pl.BlockSpec((64, 128), lambda i: (0, 0)),
          pl.BlockSpec((1, 128), lambda i: (0, 0)),
          pl.BlockSpec((128, 128), lambda i: (0, 0)),
          pl.BlockSpec((1, 128), lambda i: (0, 0)),
      ],
      out_specs=pl.BlockSpec((1024, 128), lambda i: (0, 0)),
      out_shape=jax.ShapeDtypeStruct((1024, 128), F32),
  )(l_rows, i_rows, w1a, w1b, bm1, w2p, bm2p, w3p, bm3p)


def _att_proj(a):
  Hh, C = a.shape
  if Hh == 1:
    return jnp.repeat(a.reshape(C, 1), 128, axis=1)
  M = jnp.zeros((Hh, C, Hh), F32)
  M = M.at[jnp.arange(Hh), :, jnp.arange(Hh)].set(a)
  M = M.reshape(Hh * C, Hh)
  return jnp.repeat(M, 16, axis=1)


def kernel(x, edge_index, liquor_idx, ingredient_idx,
           W1, a_src1, a_dst1, b1, W2, a_src2, a_dst2, b2,
           W3, a_src3, a_dst3, b3, Wm1, bm1, Wm2, bm2, Wm3, bm3):
  xp = jnp.zeros((NPAD, D), F32).at[:N].set(x)
  loop = jnp.arange(N, dtype=I32)
  padE = jnp.full((EPAD - E,), N, I32)
  src = jnp.concatenate([edge_index[0].astype(I32), loop, padE])
  dst = jnp.concatenate([edge_index[1].astype(I32), loop, padE])
  z128 = jnp.zeros((RPT, 128), F32)

  ws1, wd1 = _att_proj(a_src1), _att_proj(a_dst1)
  ws2, wd2 = _att_proj(a_src2), _att_proj(a_dst2)
  ws3, wd3 = _att_proj(a_src3), _att_proj(a_dst3)
  b1r = b1.reshape(1, F)
  b2r = b2.reshape(1, F)
  b3p = jnp.zeros((1, 128), F32).at[0, :64].set(b3)
  w1a = jnp.zeros((128, 64), F32).at[:64].set(Wm1[:64])
  w1b = jnp.zeros((128, 64), F32).at[:64].set(Wm1[64:])
  bm1r = bm1.reshape(1, 64)
  w2p = jnp.zeros((64, 128), F32).at[:, :32].set(Wm2)
  bm2p = jnp.zeros((1, 128), F32).at[0, :32].set(bm2)
  w3p = jnp.zeros((128, 128), F32).at[:32, 0:1].set(Wm3)
  bm3p = jnp.zeros((1, 128), F32).at[0, 0:1].set(bm3)

  h0, h1, h2, h3c, as1, ad1 = _lift1(xp, W1, ws1, wd1)
  exs1, den1 = _phase1(as1, ad1, src, dst, z128, 4)
  raw1 = _phase2((h0, h1, h2, h3c), exs1, src, dst, z128)

  g0, g1, g2, g3, as2, ad2 = _comb_lift(raw1, den1, b1r, W2, ws2, wd2, F)
  exs2, den2 = _phase1(as2, ad2, src, dst, z128, 4)
  raw2 = _phase2((g0, g1, g2, g3), exs2, src, dst, z128)

  h3pre, as3, ad3 = _comb_lift(raw2, den2, b2r, W3, ws3, wd3, 64)
  exs3, den3 = _phase1(as3, ad3, src, dst, z128, 1)
  raw3 = _phase2_chunk(h3pre, exs3[0], src, dst, z128)
  h3 = _comb3(raw3, den3, b3p)

  l_rows, i_rows = _pair_gather(h3, liquor_idx.astype(I32),
                                ingredient_idx.astype(I32))
  out = _mlp(l_rows, i_rows, w1a, w1b, bm1r, w2p, bm2p, w3p, bm3p)
  return out[:, 0]


# speedup vs baseline: 10.5134x; 1.5133x over previous
import functools

import jax
import jax.numpy as jnp
from jax import lax
from jax.experimental import pallas as pl
from jax.experimental.pallas import tpu as pltpu
from jax.experimental.pallas import tpu_sc as plsc

F32 = jnp.float32
I32 = jnp.int32

N = 10000
NPAD = 10240
D = 128
HID = 64
HEADS = 8
F = 512
E = 330000
B = 96
B1 = 64
NC, NS = 2, 16
NW = NC * NS
NB2 = 108
Q1 = NB2 * B
NB1 = Q1 // B1
EPAD = NW * Q1
NSC = 10112
RPT = NSC // NS

_mesh = plsc.VectorSubcoreMesh(
    core_axis_name="c", subcore_axis_name="s", num_cores=NC, num_subcores=NS)


def _p1_body(nex, *args):
  (as_hbm, ad_hbm, src_hbm, dst_hbm, z128_hbm) = args[:5]
  ex_outs = args[5:5 + nex]
  den_hbm = args[5 + nex]
  scr = args[6 + nex:]
  den_sh, sidx_v, didx_v, as_v, ad_v = scr[:5]
  exc_vs = scr[5:5 + nex]
  sem = scr[5 + nex]

  cid = lax.axis_index("c")
  tid = lax.axis_index("s")
  wid = tid * NC + cid
  pltpu.sync_copy(z128_hbm.at[pl.ds(0, RPT)], den_sh.at[pl.ds(tid * RPT, RPT)])
  plsc.subcore_barrier()
  base = wid * Q1

  def batch(bi, _):
    e0 = base + bi * B1
    pltpu.sync_copy(src_hbm.at[pl.ds(e0, B1)], sidx_v)
    pltpu.sync_copy(dst_hbm.at[pl.ds(e0, B1)], didx_v)
    pltpu.async_copy(as_hbm.at[sidx_v], as_v, sem).wait()
    pltpu.async_copy(ad_hbm.at[didx_v], ad_v, sem).wait()

    def edge(i, _):
      for g in range(2 * nex):
        e16 = as_v[i, pl.ds(16 * g, 16)] + ad_v[i, pl.ds(16 * g, 16)]
        e16 = jnp.where(e16 >= 0.0, e16, e16 * 0.2)
        ex = jnp.exp(e16)
        as_v[i, pl.ds(16 * g, 16)] = ex
        exc_vs[g // 2][i, pl.ds(16 * (g % 2), 16)] = ex
      return 0

    lax.fori_loop(0, B1, edge, 0, unroll=2)
    for c in range(nex):
      pltpu.sync_copy(exc_vs[c], ex_outs[c].at[pl.ds(e0, B1)])
    pltpu.sync_copy(as_v, den_sh.at[didx_v], add=True)
    return 0

  lax.fori_loop(0, NB1, batch, 0)
  plsc.subcore_barrier()
  pltpu.sync_copy(den_sh.at[pl.ds(tid * RPT, RPT)],
                  den_hbm.at[cid, pl.ds(tid * RPT, RPT)])


def _phase1(as_tab, ad_tab, src, dst, z128, nex):
  fn = pl.kernel(
      functools.partial(_p1_body, nex),
      out_type=(tuple(jax.ShapeDtypeStruct((EPAD, 32), F32)
                      for _ in range(nex))
                + (jax.ShapeDtypeStruct((NC, NPAD, 128), F32),)),
      mesh=_mesh,
      scratch_types=[
          pltpu.VMEM_SHARED((NSC, 128), F32),
          pltpu.VMEM((B1,), I32),
          pltpu.VMEM((B1,), I32),
          pltpu.VMEM((B1, 128), F32),
          pltpu.VMEM((B1, 128), F32),
      ] + [pltpu.VMEM((B1, 32), F32) for _ in range(nex)]
      + [pltpu.SemaphoreType.DMA],
  )
  outs = fn(as_tab, ad_tab, src, dst, z128)
  return outs[:nex], outs[nex]


def _p2_body(h_hbm, exc_hbm, src_hbm, dst_hbm, z128_hbm,
             raw_hbm,
             acc_sh, sidx0, sidx1, didx0, didx1, ex0, ex1, g0, g1,
             sem0, sem1):
  cid = lax.axis_index("c")
  tid = lax.axis_index("s")
  wid = tid * NC + cid
  pltpu.sync_copy(z128_hbm.at[pl.ds(0, RPT)], acc_sh.at[pl.ds(tid * RPT, RPT)])
  plsc.subcore_barrier()
  base = wid * Q1
  sidx = (sidx0, sidx1)
  didx = (didx0, didx1)
  exv = (ex0, ex1)
  gv = (g0, g1)
  sems = (sem0, sem1)

  def load_linear(b, buf):
    e0 = base + b * B
    pltpu.sync_copy(src_hbm.at[pl.ds(e0, B)], sidx[buf])
    pltpu.sync_copy(dst_hbm.at[pl.ds(e0, B)], didx[buf])
    pltpu.sync_copy(exc_hbm.at[pl.ds(e0, B)], exv[buf])

  load_linear(0, 0)
  pltpu.async_copy(h_hbm.at[sidx[0]], gv[0], sems[0])
  load_linear(1, 1)
  pltpu.async_copy(h_hbm.at[sidx[1]], gv[1], sems[1])

  def step(b, buf):
    g_v = gv[buf]
    ex_v = exv[buf]
    pltpu.make_async_copy(h_hbm.at[sidx[buf]], g_v, sems[buf]).wait()

    def edge(i, _):
      s0 = ex_v[i, pl.ds(0, 16)]
      s1 = ex_v[i, pl.ds(16, 16)]
      for j in range(8):
        s = s0 if j < 4 else s1
        g_v[i, pl.ds(j * 16, 16)] = g_v[i, pl.ds(j * 16, 16)] * s
      return 0

    lax.fori_loop(0, B, edge, 0, unroll=2)
    pltpu.sync_copy(g_v, acc_sh.at[didx[buf]], add=True)

    @pl.when(b + 2 < NB2)
    def _():
      load_linear(b + 2, buf)
      pltpu.async_copy(h_hbm.at[sidx[buf]], g_v, sems[buf])

  def loop2(it, _):
    step(2 * it, 0)
    step(2 * it + 1, 1)
    return 0

  lax.fori_loop(0, NB2 // 2, loop2, 0)
  plsc.subcore_barrier()
  pltpu.sync_copy(acc_sh.at[pl.ds(tid * RPT, RPT)],
                  raw_hbm.at[cid, pl.ds(tid * RPT, RPT)])


def _phase2_chunk(h_chunk, exc, src, dst, z128):
  fn = pl.kernel(
      _p2_body,
      out_type=jax.ShapeDtypeStruct((NC, NPAD, 128), F32),
      mesh=_mesh,
      scratch_types=[
          pltpu.VMEM_SHARED((NSC, 128), F32),
          pltpu.VMEM((B,), I32),
          pltpu.VMEM((B,), I32),
          pltpu.VMEM((B,), I32),
          pltpu.VMEM((B,), I32),
          pltpu.VMEM((B, 32), F32),
          pltpu.VMEM((B, 32), F32),
          pltpu.VMEM((B, 128), F32),
          pltpu.VMEM((B, 128), F32),
          pltpu.SemaphoreType.DMA,
          pltpu.SemaphoreType.DMA,
      ],
  )
  return fn(h_chunk, exc, src, dst, z128)


def _phase2(h_chunks, exs, src, dst, z128):
  return [_phase2_chunk(h_chunks[c], exs[c], src, dst, z128)
          for c in range(4)]


def _pair_body(h3_hbm, liq_hbm, ing_hbm, l_out, i_out, idx_v, rows_v, sem):
  cid = lax.axis_index("c")
  tid = lax.axis_index("s")
  wid = tid * NC + cid
  base = wid * 32
  pltpu.sync_copy(liq_hbm.at[pl.ds(base, 32)], idx_v)
  pltpu.async_copy(h3_hbm.at[idx_v], rows_v, sem).wait()
  pltpu.sync_copy(rows_v, l_out.at[pl.ds(base, 32)])
  pltpu.sync_copy(ing_hbm.at[pl.ds(base, 32)], idx_v)
  pltpu.async_copy(h3_hbm.at[idx_v], rows_v, sem).wait()
  pltpu.sync_copy(rows_v, i_out.at[pl.ds(base, 32)])


def _pair_gather(h3, liq, ing):
  fn = pl.kernel(
      _pair_body,
      out_type=(jax.ShapeDtypeStruct((1024, 128), F32),
                jax.ShapeDtypeStruct((1024, 128), F32)),
      mesh=_mesh,
      scratch_types=[
          pltpu.VMEM((32,), I32),
          pltpu.VMEM((32, 128), F32),
          pltpu.SemaphoreType.DMA,
      ],
  )
  return fn(h3, liq, ing)


_BLK = 256
_GRID = NPAD // _BLK


def _lift1_body(x_ref, w_ref, ws_ref, wd_ref, h0, h1, h2, h3, as_ref, ad_ref):
  h = jnp.dot(x_ref[...], w_ref[...], preferred_element_type=F32)
  h0[...] = h[:, 0:128]
  h1[...] = h[:, 128:256]
  h2[...] = h[:, 256:384]
  h3[...] = h[:, 384:512]
  as_ref[...] = jnp.dot(h, ws_ref[...], preferred_element_type=F32)
  ad_ref[...] = jnp.dot(h, wd_ref[...], preferred_element_type=F32)


def _lift1(xp, W1, ws, wd):
  return pl.pallas_call(
      _lift1_body,
      grid=(_GRID,),
      in_specs=[
          pl.BlockSpec((_BLK, D), lambda i: (i, 0)),
          pl.BlockSpec((D, F), lambda i: (0, 0)),
          pl.BlockSpec((F, 128), lambda i: (0, 0)),
          pl.BlockSpec((F, 128), lambda i: (0, 0)),
      ],
      out_specs=[pl.BlockSpec((_BLK, 128), lambda i: (i, 0))] * 6,
      out_shape=[jax.ShapeDtypeStruct((NPAD, 128), F32)] * 6,
  )(xp, W1, ws, wd)


def _comb_act(r_refs, den_ref, b_ref):
  d = den_ref[0] + den_ref[1]
  inv = 1.0 / (d + 1e-16)
  cols = []
  for h in range(HEADS):
    rr = r_refs[h // 2]
    blk = (rr[0] + rr[1])[:, (h % 2) * 64:(h % 2) * 64 + 64]
    cols.append(blk * inv[:, 16 * h:16 * h + 1])
  act = jnp.concatenate(cols, axis=1) + b_ref[...]
  return jnp.maximum(act, 0.0)


def _comb_lift_body(cw, r0, r1, r2, r3, den_ref, b_ref, w_ref,
                    ws_ref, wd_ref, *outs):
  act = _comb_act([r0, r1, r2, r3], den_ref, b_ref)
  h = jnp.dot(act, w_ref[...], preferred_element_type=F32)
  nchunks = cw * 0 + (F // 128 if cw == 128 else 1)
  hc = h
  if cw < 128:
    hc = jnp.concatenate([h, jnp.zeros((h.shape[0], 128 - cw), F32)], axis=1)
  for c in range(nchunks):
    outs[c][...] = hc[:, c * 128:c * 128 + 128] if cw == 128 else hc
  as_out = jnp.dot(h, ws_ref[...], preferred_element_type=F32)
  ad_out = jnp.dot(h, wd_ref[...], preferred_element_type=F32)
  outs[nchunks][...] = as_out
  outs[nchunks + 1][...] = ad_out


def _comb_lift(raws, den, b, W, ws, wd, fout):
  nchunks = 4 if fout == F else 1
  cw = fout // nchunks
  body = functools.partial(_comb_lift_body, cw)
  return pl.pallas_call(
      body,
      grid=(_GRID,),
      in_specs=[pl.BlockSpec((NC, _BLK, 128), lambda i: (0, i, 0))] * 4
      + [
          pl.BlockSpec((NC, _BLK, 128), lambda i: (0, i, 0)),
          pl.BlockSpec((1, F), lambda i: (0, 0)),
          pl.BlockSpec((F, fout), lambda i: (0, 0)),
          pl.BlockSpec((fout, 128), lambda i: (0, 0)),
          pl.BlockSpec((fout, 128), lambda i: (0, 0)),
      ],
      out_specs=[pl.BlockSpec((_BLK, 128), lambda i: (i, 0))] * (nchunks + 2),
      out_shape=[jax.ShapeDtypeStruct((NPAD, 128), F32)] * (nchunks + 2),
  )(*raws, den, b, W, ws, wd)


def _comb3_body(r_ref, den_ref, b_ref, out_ref):
  d = den_ref[0] + den_ref[1]
  inv = 1.0 / (d[:, 0:1] + 1e-16)
  out_ref[...] = (r_ref[0] + r_ref[1]) * inv + b_ref[...]


def _comb3(raw3, den3, b3p):
  return pl.pallas_call(
      _comb3_body,
      grid=(_GRID,),
      in_specs=[
          pl.BlockSpec((NC, _BLK, 128), lambda i: (0, i, 0)),
          pl.BlockSpec((NC, _BLK, 128), lambda i: (0, i, 0)),
          pl.BlockSpec((1, 128), lambda i: (0, 0)),
      ],
      out_specs=pl.BlockSpec((_BLK, 128), lambda i: (i, 0)),
      out_shape=jax.ShapeDtypeStruct((NPAD, 128), F32),
  )(raw3, den3, b3p)


def _mlp_body(l_ref, i_ref, w1a_ref, w1b_ref, b1_ref, w2_ref, b2_ref,
              w3_ref, b3_ref, out_ref):
  z = (jnp.dot(l_ref[...], w1a_ref[...], preferred_element_type=F32)
       + jnp.dot(i_ref[...], w1b_ref[...], preferred_element_type=F32)
       + b1_ref[...])
  z = jnp.maximum(z, 0.0)
  z = jnp.dot(z, w2_ref[...], preferred_element_type=F32) + b2_ref[...]
  z = jnp.maximum(z, 0.0)
  z = jnp.dot(z, w3_ref[...], preferred_element_type=F32) + b3_ref[...]
  out_ref[...] = 1.0 / (1.0 + jnp.exp(-z))


def _mlp(l_rows, i_rows, w1a, w1b, bm1, w2p, bm2p, w3p, bm3p):
  return pl.pallas_call(
      _mlp_body,
      grid=(1,),
      in_specs=[
          pl.BlockSpec((1024, 128), lambda i: (0, 0)),
          pl.BlockSpec((1024, 128), lambda i: (0, 0)),
          pl.BlockSpec((128, 64), lambda i: (0, 0)),
          pl.BlockSpec((128, 64), lambda i: (0, 0)),
          pl.BlockSpec((1, 64), lambda i: (0, 0)),
          pl.BlockSpec((64, 128), lambda i: (0, 0)),
          pl.BlockSpec((1, 128), lambda i: (0, 0)),
          pl.BlockSpec((128, 128), lambda i: (0, 0)),
          pl.BlockSpec((1, 128), lambda i: (0, 0)),
      ],
      out_specs=pl.BlockSpec((1024, 128), lambda i: (0, 0)),
      out_shape=jax.ShapeDtypeStruct((1024, 128), F32),
  )(l_rows, i_rows, w1a, w1b, bm1, w2p, bm2p, w3p, bm3p)


def _att_proj(a):
  Hh, C = a.shape
  if Hh == 1:
    return jnp.repeat(a.reshape(C, 1), 128, axis=1)
  M = jnp.zeros((Hh, C, Hh), F32)
  M = M.at[jnp.arange(Hh), :, jnp.arange(Hh)].set(a)
  M = M.reshape(Hh * C, Hh)
  return jnp.repeat(M, 16, axis=1)


def kernel(x, edge_index, liquor_idx, ingredient_idx,
           W1, a_src1, a_dst1, b1, W2, a_src2, a_dst2, b2,
           W3, a_src3, a_dst3, b3, Wm1, bm1, Wm2, bm2, Wm3, bm3):
  xp = jnp.zeros((NPAD, D), F32).at[:N].set(x)
  loop = jnp.arange(N, dtype=I32)
  padE = jnp.full((EPAD - E,), N, I32)
  src = jnp.concatenate([edge_index[0].astype(I32), loop, padE])
  dst = jnp.concatenate([edge_index[1].astype(I32), loop, padE])
  z128 = jnp.zeros((RPT, 128), F32)

  ws1, wd1 = _att_proj(a_src1), _att_proj(a_dst1)
  ws2, wd2 = _att_proj(a_src2), _att_proj(a_dst2)
  ws3, wd3 = _att_proj(a_src3), _att_proj(a_dst3)
  b1r = b1.reshape(1, F)
  b2r = b2.reshape(1, F)
  b3p = jnp.zeros((1, 128), F32).at[0, :64].set(b3)
  w1a = jnp.zeros((128, 64), F32).at[:64].set(Wm1[:64])
  w1b = jnp.zeros((128, 64), F32).at[:64].set(Wm1[64:])
  bm1r = bm1.reshape(1, 64)
  w2p = jnp.zeros((64, 128), F32).at[:, :32].set(Wm2)
  bm2p = jnp.zeros((1, 128), F32).at[0, :32].set(bm2)
  w3p = jnp.zeros((128, 128), F32).at[:32, 0:1].set(Wm3)
  bm3p = jnp.zeros((1, 128), F32).at[0, 0:1].set(bm3)

  h0, h1, h2, h3c, as1, ad1 = _lift1(xp, W1, ws1, wd1)
  exs1, den1 = _phase1(as1, ad1, src, dst, z128, 4)
  raw1 = _phase2((h0, h1, h2, h3c), exs1, src, dst, z128)

  g0, g1, g2, g3, as2, ad2 = _comb_lift(raw1, den1, b1r, W2, ws2, wd2, F)
  exs2, den2 = _phase1(as2, ad2, src, dst, z128, 4)
  raw2 = _phase2((g0, g1, g2, g3), exs2, src, dst, z128)

  h3pre, as3, ad3 = _comb_lift(raw2, den2, b2r, W3, ws3, wd3, 64)
  exs3, den3 = _phase1(as3, ad3, src, dst, z128, 1)
  raw3 = _phase2_chunk(h3pre, exs3[0], src, dst, z128)
  h3 = _comb3(raw3, den3, b3p)

  l_rows, i_rows = _pair_gather(h3, liquor_idx.astype(I32),
                                ingredient_idx.astype(I32))
  out = _mlp(l_rows, i_rows, w1a, w1b, bm1r, w2p, bm2p, w3p, bm3p)
  return out[:, 0]


# speedup vs baseline: 12.7551x; 1.2132x over previous
import functools

import jax
import jax.numpy as jnp
from jax import lax
from jax.experimental import pallas as pl
from jax.experimental.pallas import tpu as pltpu
from jax.experimental.pallas import tpu_sc as plsc

F32 = jnp.float32
I32 = jnp.int32

N = 10000
NPAD = 10240
D = 128
HID = 64
HEADS = 8
F = 512
E = 330000
B = 96
B1 = 48
NC, NS = 2, 16
NW = NC * NS
NB2 = 108
Q1 = NB2 * B
NB1 = Q1 // B1
EPAD = NW * Q1
NSC = 10112
RPT = NSC // NS

_mesh = plsc.VectorSubcoreMesh(
    core_axis_name="c", subcore_axis_name="s", num_cores=NC, num_subcores=NS)


def _p1_body(nex, *args):
  (as_hbm, ad_hbm, src_hbm, dst_hbm, z128_hbm) = args[:5]
  ex_outs = args[5:5 + nex]
  den_hbm = args[5 + nex]
  scr = args[6 + nex:]
  den_sh = scr[0]
  sidx = scr[1:3]
  didx = scr[3:5]
  asv = scr[5:7]
  adv = scr[7:9]
  exc_vs = scr[9:9 + nex]
  sems = scr[9 + nex:11 + nex]

  cid = lax.axis_index("c")
  tid = lax.axis_index("s")
  wid = tid * NC + cid
  pltpu.sync_copy(z128_hbm.at[pl.ds(0, RPT)], den_sh.at[pl.ds(tid * RPT, RPT)])
  plsc.subcore_barrier()
  base = wid * Q1

  def load_and_gather(b, buf):
    e0 = base + b * B1
    pltpu.sync_copy(src_hbm.at[pl.ds(e0, B1)], sidx[buf])
    pltpu.sync_copy(dst_hbm.at[pl.ds(e0, B1)], didx[buf])
    pltpu.async_copy(as_hbm.at[sidx[buf]], asv[buf], sems[buf])
    pltpu.async_copy(ad_hbm.at[didx[buf]], adv[buf], sems[buf])

  load_and_gather(0, 0)
  load_and_gather(1, 1)

  def step(b, buf):
    as_v = asv[buf]
    ad_v = adv[buf]
    pltpu.make_async_copy(as_hbm.at[sidx[buf]], as_v, sems[buf]).wait()
    pltpu.make_async_copy(ad_hbm.at[didx[buf]], ad_v, sems[buf]).wait()
    e0 = base + b * B1

    def edge(i, _):
      for g in range(2 * nex):
        e16 = as_v[i, pl.ds(16 * g, 16)] + ad_v[i, pl.ds(16 * g, 16)]
        e16 = jnp.where(e16 >= 0.0, e16, e16 * 0.2)
        ex = jnp.exp(e16)
        as_v[i, pl.ds(16 * g, 16)] = ex
        exc_vs[g // 2][i, pl.ds(16 * (g % 2), 16)] = ex
      return 0

    lax.fori_loop(0, B1, edge, 0, unroll=2)
    for c in range(nex):
      pltpu.sync_copy(exc_vs[c], ex_outs[c].at[pl.ds(e0, B1)])
    pltpu.sync_copy(as_v, den_sh.at[didx[buf]], add=True)

    @pl.when(b + 2 < NB1)
    def _():
      load_and_gather(b + 2, buf)

  def loop2(it, _):
    step(2 * it, 0)
    step(2 * it + 1, 1)
    return 0

  lax.fori_loop(0, NB1 // 2, loop2, 0)
  plsc.subcore_barrier()
  pltpu.sync_copy(den_sh.at[pl.ds(tid * RPT, RPT)],
                  den_hbm.at[cid, pl.ds(tid * RPT, RPT)])


def _phase1(as_tab, ad_tab, src, dst, z128, nex):
  fn = pl.kernel(
      functools.partial(_p1_body, nex),
      out_type=(tuple(jax.ShapeDtypeStruct((EPAD, 32), F32)
                      for _ in range(nex))
                + (jax.ShapeDtypeStruct((NC, NPAD, 128), F32),)),
      mesh=_mesh,
      scratch_types=[
          pltpu.VMEM_SHARED((NSC, 128), F32),
          pltpu.VMEM((B1,), I32),
          pltpu.VMEM((B1,), I32),
          pltpu.VMEM((B1,), I32),
          pltpu.VMEM((B1,), I32),
          pltpu.VMEM((B1, 128), F32),
          pltpu.VMEM((B1, 128), F32),
          pltpu.VMEM((B1, 128), F32),
          pltpu.VMEM((B1, 128), F32),
      ] + [pltpu.VMEM((B1, 32), F32) for _ in range(nex)]
      + [pltpu.SemaphoreType.DMA, pltpu.SemaphoreType.DMA],
  )
  outs = fn(as_tab, ad_tab, src, dst, z128)
  return outs[:nex], outs[nex]


def _p2_body(h_hbm, exc_hbm, src_hbm, dst_hbm, z128_hbm,
             raw_hbm,
             acc_sh, sidx0, sidx1, didx0, didx1, ex0, ex1, g0, g1,
             sem0, sem1, ssem0, ssem1):
  cid = lax.axis_index("c")
  tid = lax.axis_index("s")
  wid = tid * NC + cid
  pltpu.sync_copy(z128_hbm.at[pl.ds(0, RPT)], acc_sh.at[pl.ds(tid * RPT, RPT)])
  plsc.subcore_barrier()
  base = wid * Q1
  sidx = (sidx0, sidx1)
  didx = (didx0, didx1)
  exv = (ex0, ex1)
  gv = (g0, g1)
  sems = (sem0, sem1)
  ssems = (ssem0, ssem1)

  def load_linear(b, buf):
    e0 = base + b * B
    pltpu.sync_copy(src_hbm.at[pl.ds(e0, B)], sidx[buf])
    pltpu.sync_copy(dst_hbm.at[pl.ds(e0, B)], didx[buf])
    pltpu.sync_copy(exc_hbm.at[pl.ds(e0, B)], exv[buf])

  load_linear(0, 0)
  pltpu.async_copy(h_hbm.at[sidx[0]], gv[0], sems[0])
  load_linear(1, 1)
  pltpu.async_copy(h_hbm.at[sidx[1]], gv[1], sems[1])

  def step(b, buf):
    g_v = gv[buf]
    ex_v = exv[buf]
    pltpu.make_async_copy(h_hbm.at[sidx[buf]], g_v, sems[buf]).wait()

    def edge(i, _):
      s0 = ex_v[i, pl.ds(0, 16)]
      s1 = ex_v[i, pl.ds(16, 16)]
      for j in range(8):
        s = s0 if j < 4 else s1
        g_v[i, pl.ds(j * 16, 16)] = g_v[i, pl.ds(j * 16, 16)] * s
      return 0

    lax.fori_loop(0, B, edge, 0, unroll=2)
    e2 = base + (b + 2) * B
    scat = pltpu.async_copy(g_v, acc_sh.at[didx[buf]], ssems[buf], add=True)

    @pl.when(b + 2 < NB2)
    def _():
      pltpu.sync_copy(src_hbm.at[pl.ds(e2, B)], sidx[buf])
      pltpu.sync_copy(exc_hbm.at[pl.ds(e2, B)], exv[buf])

    scat.wait()

    @pl.when(b + 2 < NB2)
    def _():
      pltpu.sync_copy(dst_hbm.at[pl.ds(e2, B)], didx[buf])
      pltpu.async_copy(h_hbm.at[sidx[buf]], g_v, sems[buf])

  def loop2(it, _):
    step(2 * it, 0)
    step(2 * it + 1, 1)
    return 0

  lax.fori_loop(0, NB2 // 2, loop2, 0)
  plsc.subcore_barrier()
  pltpu.sync_copy(acc_sh.at[pl.ds(tid * RPT, RPT)],
                  raw_hbm.at[cid, pl.ds(tid * RPT, RPT)])


def _phase2_chunk(h_chunk, exc, src, dst, z128):
  fn = pl.kernel(
      _p2_body,
      out_type=jax.ShapeDtypeStruct((NC, NPAD, 128), F32),
      mesh=_mesh,
      scratch_types=[
          pltpu.VMEM_SHARED((NSC, 128), F32),
          pltpu.VMEM((B,), I32),
          pltpu.VMEM((B,), I32),
          pltpu.VMEM((B,), I32),
          pltpu.VMEM((B,), I32),
          pltpu.VMEM((B, 32), F32),
          pltpu.VMEM((B, 32), F32),
          pltpu.VMEM((B, 128), F32),
          pltpu.VMEM((B, 128), F32),
          pltpu.SemaphoreType.DMA,
          pltpu.SemaphoreType.DMA,
          pltpu.SemaphoreType.DMA,
          pltpu.SemaphoreType.DMA,
      ],
  )
  return fn(h_chunk, exc, src, dst, z128)


def _phase2(h_chunks, exs, src, dst, z128):
  return [_phase2_chunk(h_chunks[c], exs[c], src, dst, z128)
          for c in range(4)]


def _pair_body(h3_hbm, liq_hbm, ing_hbm, l_out, i_out, idx_v, rows_v, sem):
  cid = lax.axis_index("c")
  tid = lax.axis_index("s")
  wid = tid * NC + cid
  base = wid * 32
  pltpu.sync_copy(liq_hbm.at[pl.ds(base, 32)], idx_v)
  pltpu.async_copy(h3_hbm.at[idx_v], rows_v, sem).wait()
  pltpu.sync_copy(rows_v, l_out.at[pl.ds(base, 32)])
  pltpu.sync_copy(ing_hbm.at[pl.ds(base, 32)], idx_v)
  pltpu.async_copy(h3_hbm.at[idx_v], rows_v, sem).wait()
  pltpu.sync_copy(rows_v, i_out.at[pl.ds(base, 32)])


def _pair_gather(h3, liq, ing):
  fn = pl.kernel(
      _pair_body,
      out_type=(jax.ShapeDtypeStruct((1024, 128), F32),
                jax.ShapeDtypeStruct((1024, 128), F32)),
      mesh=_mesh,
      scratch_types=[
          pltpu.VMEM((32,), I32),
          pltpu.VMEM((32, 128), F32),
          pltpu.SemaphoreType.DMA,
      ],
  )
  return fn(h3, liq, ing)


_BLK = 256
_GRID = NPAD // _BLK


def _lift1_body(x_ref, w_ref, ws_ref, wd_ref, h0, h1, h2, h3, as_ref, ad_ref):
  h = jnp.dot(x_ref[...], w_ref[...], preferred_element_type=F32)
  h0[...] = h[:, 0:128]
  h1[...] = h[:, 128:256]
  h2[...] = h[:, 256:384]
  h3[...] = h[:, 384:512]
  as_ref[...] = jnp.dot(h, ws_ref[...], preferred_element_type=F32)
  ad_ref[...] = jnp.dot(h, wd_ref[...], preferred_element_type=F32)


def _lift1(xp, W1, ws, wd):
  return pl.pallas_call(
      _lift1_body,
      grid=(_GRID,),
      in_specs=[
          pl.BlockSpec((_BLK, D), lambda i: (i, 0)),
          pl.BlockSpec((D, F), lambda i: (0, 0)),
          pl.BlockSpec((F, 128), lambda i: (0, 0)),
          pl.BlockSpec((F, 128), lambda i: (0, 0)),
      ],
      out_specs=[pl.BlockSpec((_BLK, 128), lambda i: (i, 0))] * 6,
      out_shape=[jax.ShapeDtypeStruct((NPAD, 128), F32)] * 6,
  )(xp, W1, ws, wd)


def _comb_act(r_refs, den_ref, b_ref):
  d = den_ref[0] + den_ref[1]
  inv = 1.0 / (d + 1e-16)
  cols = []
  for h in range(HEADS):
    rr = r_refs[h // 2]
    blk = (rr[0] + rr[1])[:, (h % 2) * 64:(h % 2) * 64 + 64]
    cols.append(blk * inv[:, 16 * h:16 * h + 1])
  act = jnp.concatenate(cols, axis=1) + b_ref[...]
  return jnp.maximum(act, 0.0)


def _comb_lift_body(cw, r0, r1, r2, r3, den_ref, b_ref, w_ref,
                    ws_ref, wd_ref, *outs):
  act = _comb_act([r0, r1, r2, r3], den_ref, b_ref)
  h = jnp.dot(act, w_ref[...], preferred_element_type=F32)
  nchunks = cw * 0 + (F // 128 if cw == 128 else 1)
  hc = h
  if cw < 128:
    hc = jnp.concatenate([h, jnp.zeros((h.shape[0], 128 - cw), F32)], axis=1)
  for c in range(nchunks):
    outs[c][...] = hc[:, c * 128:c * 128 + 128] if cw == 128 else hc
  as_out = jnp.dot(h, ws_ref[...], preferred_element_type=F32)
  ad_out = jnp.dot(h, wd_ref[...], preferred_element_type=F32)
  outs[nchunks][...] = as_out
  outs[nchunks + 1][...] = ad_out


def _comb_lift(raws, den, b, W, ws, wd, fout):
  nchunks = 4 if fout == F else 1
  cw = fout // nchunks
  body = functools.partial(_comb_lift_body, cw)
  return pl.pallas_call(
      body,
      grid=(_GRID,),
      in_specs=[pl.BlockSpec((NC, _BLK, 128), lambda i: (0, i, 0))] * 4
      + [
          pl.BlockSpec((NC, _BLK, 128), lambda i: (0, i, 0)),
          pl.BlockSpec((1, F), lambda i: (0, 0)),
          pl.BlockSpec((F, fout), lambda i: (0, 0)),
          pl.BlockSpec((fout, 128), lambda i: (0, 0)),
          pl.BlockSpec((fout, 128), lambda i: (0, 0)),
      ],
      out_specs=[pl.BlockSpec((_BLK, 128), lambda i: (i, 0))] * (nchunks + 2),
      out_shape=[jax.ShapeDtypeStruct((NPAD, 128), F32)] * (nchunks + 2),
  )(*raws, den, b, W, ws, wd)


def _comb3_body(r_ref, den_ref, b_ref, out_ref):
  d = den_ref[0] + den_ref[1]
  inv = 1.0 / (d[:, 0:1] + 1e-16)
  out_ref[...] = (r_ref[0] + r_ref[1]) * inv + b_ref[...]


def _comb3(raw3, den3, b3p):
  return pl.pallas_call(
      _comb3_body,
      grid=(_GRID,),
      in_specs=[
          pl.BlockSpec((NC, _BLK, 128), lambda i: (0, i, 0)),
          pl.BlockSpec((NC, _BLK, 128), lambda i: (0, i, 0)),
          pl.BlockSpec((1, 128), lambda i: (0, 0)),
      ],
      out_specs=pl.BlockSpec((_BLK, 128), lambda i: (i, 0)),
      out_shape=jax.ShapeDtypeStruct((NPAD, 128), F32),
  )(raw3, den3, b3p)


def _mlp_body(l_ref, i_ref, w1a_ref, w1b_ref, b1_ref, w2_ref, b2_ref,
              w3_ref, b3_ref, out_ref):
  z = (jnp.dot(l_ref[...], w1a_ref[...], preferred_element_type=F32)
       + jnp.dot(i_ref[...], w1b_ref[...], preferred_element_type=F32)
       + b1_ref[...])
  z = jnp.maximum(z, 0.0)
  z = jnp.dot(z, w2_ref[...], preferred_element_type=F32) + b2_ref[...]
  z = jnp.maximum(z, 0.0)
  z = jnp.dot(z, w3_ref[...], preferred_element_type=F32) + b3_ref[...]
  out_ref[...] = 1.0 / (1.0 + jnp.exp(-z))


def _mlp(l_rows, i_rows, w1a, w1b, bm1, w2p, bm2p, w3p, bm3p):
  return pl.pallas_call(
      _mlp_body,
      grid=(1,),
      in_specs=[
          pl.BlockSpec((1024, 128), lambda i: (0, 0)),
          pl.BlockSpec((1024, 128), lambda i: (0, 0)),
          pl.BlockSpec((128, 64), lambda i: (0, 0)),
          pl.BlockSpec((128, 64), lambda i: (0, 0)),
          pl.BlockSpec((1, 64), lambda i: (0, 0)),
          pl.BlockSpec((64, 128), lambda i: (0, 0)),
          pl.BlockSpec((1, 128), lambda i: (0, 0)),
          pl.BlockSpec((128, 128), lambda i: (0, 0)),
          pl.BlockSpec((1, 128), lambda i: (0, 0)),
      ],
      out_specs=pl.BlockSpec((1024, 128), lambda i: (0, 0)),
      out_shape=jax.ShapeDtypeStruct((1024, 128), F32),
  )(l_rows, i_rows, w1a, w1b, bm1, w2p, bm2p, w3p, bm3p)


def _att_proj(a):
  Hh, C = a.shape
  if Hh == 1:
    return jnp.repeat(a.reshape(C, 1), 128, axis=1)
  M = jnp.zeros((Hh, C, Hh), F32)
  M = M.at[jnp.arange(Hh), :, jnp.arange(Hh)].set(a)
  M = M.reshape(Hh * C, Hh)
  return jnp.repeat(M, 16, axis=1)


def kernel(x, edge_index, liquor_idx, ingredient_idx,
           W1, a_src1, a_dst1, b1, W2, a_src2, a_dst2, b2,
           W3, a_src3, a_dst3, b3, Wm1, bm1, Wm2, bm2, Wm3, bm3):
  xp = jnp.zeros((NPAD, D), F32).at[:N].set(x)
  loop = jnp.arange(N, dtype=I32)
  padE = jnp.full((EPAD - E,), N, I32)
  src = jnp.concatenate([edge_index[0].astype(I32), loop, padE])
  dst = jnp.concatenate([edge_index[1].astype(I32), loop, padE])
  z128 = jnp.zeros((RPT, 128), F32)

  ws1, wd1 = _att_proj(a_src1), _att_proj(a_dst1)
  ws2, wd2 = _att_proj(a_src2), _att_proj(a_dst2)
  ws3, wd3 = _att_proj(a_src3), _att_proj(a_dst3)
  b1r = b1.reshape(1, F)
  b2r = b2.reshape(1, F)
  b3p = jnp.zeros((1, 128), F32).at[0, :64].set(b3)
  w1a = jnp.zeros((128, 64), F32).at[:64].set(Wm1[:64])
  w1b = jnp.zeros((128, 64), F32).at[:64].set(Wm1[64:])
  bm1r = bm1.reshape(1, 64)
  w2p = jnp.zeros((64, 128), F32).at[:, :32].set(Wm2)
  bm2p = jnp.zeros((1, 128), F32).at[0, :32].set(bm2)
  w3p = jnp.zeros((128, 128), F32).at[:32, 0:1].set(Wm3)
  bm3p = jnp.zeros((1, 128), F32).at[0, 0:1].set(bm3)

  h0, h1, h2, h3c, as1, ad1 = _lift1(xp, W1, ws1, wd1)
  exs1, den1 = _phase1(as1, ad1, src, dst, z128, 4)
  raw1 = _phase2((h0, h1, h2, h3c), exs1, src, dst, z128)

  g0, g1, g2, g3, as2, ad2 = _comb_lift(raw1, den1, b1r, W2, ws2, wd2, F)
  exs2, den2 = _phase1(as2, ad2, src, dst, z128, 4)
  raw2 = _phase2((g0, g1, g2, g3), exs2, src, dst, z128)

  h3pre, as3, ad3 = _comb_lift(raw2, den2, b2r, W3, ws3, wd3, 64)
  exs3, den3 = _phase1(as3, ad3, src, dst, z128, 1)
  raw3 = _phase2_chunk(h3pre, exs3[0], src, dst, z128)
  h3 = _comb3(raw3, den3, b3p)

  l_rows, i_rows = _pair_gather(h3, liquor_idx.astype(I32),
                                ingredient_idx.astype(I32))
  out = _mlp(l_rows, i_rows, w1a, w1b, bm1r, w2p, bm2p, w3p, bm3p)
  return out[:, 0]


# speedup vs baseline: 13.5365x; 1.0613x over previous
import functools

import jax
import jax.numpy as jnp
from jax import lax
from jax.experimental import pallas as pl
from jax.experimental.pallas import tpu as pltpu
from jax.experimental.pallas import tpu_sc as plsc

F32 = jnp.float32
I32 = jnp.int32

N = 10000
NPAD = 10240
D = 128
HID = 64
HEADS = 8
F = 512
E = 330000
B = 96
B1 = 32
NC, NS = 2, 16
NW = NC * NS
NB2 = 108
Q1 = NB2 * B
NB1 = Q1 // B1
EPAD = NW * Q1
NSC = 10112
RPT = NSC // NS

_mesh = plsc.VectorSubcoreMesh(
    core_axis_name="c", subcore_axis_name="s", num_cores=NC, num_subcores=NS)


def _p1_body(nex, *args):
  (as_hbm, ad_hbm, src_hbm, dst_hbm, z128_hbm) = args[:5]
  ex_outs = args[5:5 + nex]
  den_hbm = args[5 + nex]
  scr = args[6 + nex:]
  den_sh = scr[0]
  sidx = scr[1:5]
  didx = scr[5:9]
  asv = scr[9:11]
  adv = scr[11:13]
  denrow = scr[13]
  exc_vs = scr[14:14 + nex]
  gsem = scr[14 + nex:16 + nex]
  esem = scr[16 + nex]
  ssem = scr[17 + nex]

  cid = lax.axis_index("c")
  tid = lax.axis_index("s")
  wid = tid * NC + cid
  pltpu.sync_copy(z128_hbm.at[pl.ds(0, RPT)], den_sh.at[pl.ds(tid * RPT, RPT)])
  plsc.subcore_barrier()
  base = wid * Q1

  def load_and_gather(b, ib, buf):
    e0 = base + b * B1
    pltpu.sync_copy(src_hbm.at[pl.ds(e0, B1)], sidx[ib])
    pltpu.sync_copy(dst_hbm.at[pl.ds(e0, B1)], didx[ib])
    pltpu.async_copy(as_hbm.at[sidx[ib]], asv[buf], gsem[buf])
    pltpu.async_copy(ad_hbm.at[didx[ib]], adv[buf], gsem[buf])

  def drain_exc():
    for c in range(nex):
      pltpu.make_async_copy(
          exc_vs[c], ex_outs[c].at[pl.ds(0, B1)], esem).wait()

  def drain_den():
    pltpu.make_async_copy(denrow, den_sh.at[pl.ds(0, B1)], ssem).wait()

  load_and_gather(0, 0, 0)
  load_and_gather(1, 1, 1)

  def step(b, k):
    buf = k % 2
    as_v = asv[buf]
    ad_v = adv[buf]
    dr_v = denrow

    @pl.when(b >= 1)
    def _():
      drain_exc()
      drain_den()

    pltpu.make_async_copy(as_hbm.at[sidx[k]], as_v, gsem[buf]).wait()
    pltpu.make_async_copy(ad_hbm.at[didx[k]], ad_v, gsem[buf]).wait()
    e0 = base + b * B1

    def edge(i, _):
      for g in range(2 * nex):
        e16 = as_v[i, pl.ds(16 * g, 16)] + ad_v[i, pl.ds(16 * g, 16)]
        e16 = jnp.where(e16 >= 0.0, e16, e16 * 0.2)
        ex = jnp.exp(e16)
        dr_v[i, pl.ds(16 * g, 16)] = ex
        exc_vs[g // 2][i, pl.ds(16 * (g % 2), 16)] = ex
      return 0

    lax.fori_loop(0, B1, edge, 0, unroll=2)
    for c in range(nex):
      pltpu.async_copy(exc_vs[c], ex_outs[c].at[pl.ds(e0, B1)], esem)
    pltpu.async_copy(dr_v, den_sh.at[didx[k]], ssem, add=True)

    @pl.when(b + 2 < NB1)
    def _():
      load_and_gather(b + 2, (k + 2) % 4, buf)

  def loop4(it, _):
    b0 = 4 * it
    for k in range(4):
      step(b0 + k, k)
    return 0

  lax.fori_loop(0, NB1 // 4, loop4, 0)
  drain_exc()
  drain_den()
  plsc.subcore_barrier()
  pltpu.sync_copy(den_sh.at[pl.ds(tid * RPT, RPT)],
                  den_hbm.at[cid, pl.ds(tid * RPT, RPT)])


def _phase1(as_tab, ad_tab, src, dst, z128, nex):
  fn = pl.kernel(
      functools.partial(_p1_body, nex),
      out_type=(tuple(jax.ShapeDtypeStruct((EPAD, 32), F32)
                      for _ in range(nex))
                + (jax.ShapeDtypeStruct((NC, NPAD, 128), F32),)),
      mesh=_mesh,
      scratch_types=[
          pltpu.VMEM_SHARED((NSC, 128), F32),
      ] + [pltpu.VMEM((B1,), I32) for _ in range(8)]
      + [pltpu.VMEM((B1, 128), F32) for _ in range(5)]
      + [pltpu.VMEM((B1, 32), F32) for _ in range(nex)]
      + [pltpu.SemaphoreType.DMA for _ in range(4)],
  )
  outs = fn(as_tab, ad_tab, src, dst, z128)
  return outs[:nex], outs[nex]


def _p2_body(h_hbm, exc_hbm, src_hbm, dst_hbm, z128_hbm,
             raw_hbm,
             acc_sh, sidx0, sidx1, didx0, didx1, ex0, ex1, g0, g1,
             sem0, sem1, ssem0, ssem1):
  cid = lax.axis_index("c")
  tid = lax.axis_index("s")
  wid = tid * NC + cid
  pltpu.sync_copy(z128_hbm.at[pl.ds(0, RPT)], acc_sh.at[pl.ds(tid * RPT, RPT)])
  plsc.subcore_barrier()
  base = wid * Q1
  sidx = (sidx0, sidx1)
  didx = (didx0, didx1)
  exv = (ex0, ex1)
  gv = (g0, g1)
  sems = (sem0, sem1)
  ssems = (ssem0, ssem1)

  def load_linear(b, buf):
    e0 = base + b * B
    pltpu.sync_copy(src_hbm.at[pl.ds(e0, B)], sidx[buf])
    pltpu.sync_copy(dst_hbm.at[pl.ds(e0, B)], didx[buf])
    pltpu.sync_copy(exc_hbm.at[pl.ds(e0, B)], exv[buf])

  load_linear(0, 0)
  pltpu.async_copy(h_hbm.at[sidx[0]], gv[0], sems[0])
  load_linear(1, 1)
  pltpu.async_copy(h_hbm.at[sidx[1]], gv[1], sems[1])

  def step(b, buf):
    g_v = gv[buf]
    ex_v = exv[buf]
    pltpu.make_async_copy(h_hbm.at[sidx[buf]], g_v, sems[buf]).wait()

    def edge(i, _):
      s0 = ex_v[i, pl.ds(0, 16)]
      s1 = ex_v[i, pl.ds(16, 16)]
      for j in range(8):
        s = s0 if j < 4 else s1
        g_v[i, pl.ds(j * 16, 16)] = g_v[i, pl.ds(j * 16, 16)] * s
      return 0

    lax.fori_loop(0, B, edge, 0, unroll=2)
    e2 = base + (b + 2) * B
    scat = pltpu.async_copy(g_v, acc_sh.at[didx[buf]], ssems[buf], add=True)

    @pl.when(b + 2 < NB2)
    def _():
      pltpu.sync_copy(src_hbm.at[pl.ds(e2, B)], sidx[buf])
      pltpu.sync_copy(exc_hbm.at[pl.ds(e2, B)], exv[buf])

    scat.wait()

    @pl.when(b + 2 < NB2)
    def _():
      pltpu.sync_copy(dst_hbm.at[pl.ds(e2, B)], didx[buf])
      pltpu.async_copy(h_hbm.at[sidx[buf]], g_v, sems[buf])

  def loop2(it, _):
    step(2 * it, 0)
    step(2 * it + 1, 1)
    return 0

  lax.fori_loop(0, NB2 // 2, loop2, 0)
  plsc.subcore_barrier()
  pltpu.sync_copy(acc_sh.at[pl.ds(tid * RPT, RPT)],
                  raw_hbm.at[cid, pl.ds(tid * RPT, RPT)])


def _phase2_chunk(h_chunk, exc, src, dst, z128):
  fn = pl.kernel(
      _p2_body,
      out_type=jax.ShapeDtypeStruct((NC, NPAD, 128), F32),
      mesh=_mesh,
      scratch_types=[
          pltpu.VMEM_SHARED((NSC, 128), F32),
          pltpu.VMEM((B,), I32),
          pltpu.VMEM((B,), I32),
          pltpu.VMEM((B,), I32),
          pltpu.VMEM((B,), I32),
          pltpu.VMEM((B, 32), F32),
          pltpu.VMEM((B, 32), F32),
          pltpu.VMEM((B, 128), F32),
          pltpu.VMEM((B, 128), F32),
          pltpu.SemaphoreType.DMA,
          pltpu.SemaphoreType.DMA,
          pltpu.SemaphoreType.DMA,
          pltpu.SemaphoreType.DMA,
      ],
  )
  return fn(h_chunk, exc, src, dst, z128)


def _phase2(h_chunks, exs, src, dst, z128):
  return [_phase2_chunk(h_chunks[c], exs[c], src, dst, z128)
          for c in range(4)]


def _pair_body(h3_hbm, liq_hbm, ing_hbm, l_out, i_out, idx_v, rows_v, sem):
  cid = lax.axis_index("c")
  tid = lax.axis_index("s")
  wid = tid * NC + cid
  base = wid * 32
  pltpu.sync_copy(liq_hbm.at[pl.ds(base, 32)], idx_v)
  pltpu.async_copy(h3_hbm.at[idx_v], rows_v, sem).wait()
  pltpu.sync_copy(rows_v, l_out.at[pl.ds(base, 32)])
  pltpu.sync_copy(ing_hbm.at[pl.ds(base, 32)], idx_v)
  pltpu.async_copy(h3_hbm.at[idx_v], rows_v, sem).wait()
  pltpu.sync_copy(rows_v, i_out.at[pl.ds(base, 32)])


def _pair_gather(h3, liq, ing):
  fn = pl.kernel(
      _pair_body,
      out_type=(jax.ShapeDtypeStruct((1024, 128), F32),
                jax.ShapeDtypeStruct((1024, 128), F32)),
      mesh=_mesh,
      scratch_types=[
          pltpu.VMEM((32,), I32),
          pltpu.VMEM((32, 128), F32),
          pltpu.SemaphoreType.DMA,
      ],
  )
  return fn(h3, liq, ing)


_BLK = 256
_GRID = NPAD // _BLK


def _lift1_body(x_ref, w_ref, ws_ref, wd_ref, h0, h1, h2, h3, as_ref, ad_ref):
  h = jnp.dot(x_ref[...], w_ref[...], preferred_element_type=F32)
  h0[...] = h[:, 0:128]
  h1[...] = h[:, 128:256]
  h2[...] = h[:, 256:384]
  h3[...] = h[:, 384:512]
  as_ref[...] = jnp.dot(h, ws_ref[...], preferred_element_type=F32)
  ad_ref[...] = jnp.dot(h, wd_ref[...], preferred_element_type=F32)


def _lift1(xp, W1, ws, wd):
  return pl.pallas_call(
      _lift1_body,
      grid=(_GRID,),
      in_specs=[
          pl.BlockSpec((_BLK, D), lambda i: (i, 0)),
          pl.BlockSpec((D, F), lambda i: (0, 0)),
          pl.BlockSpec((F, 128), lambda i: (0, 0)),
          pl.BlockSpec((F, 128), lambda i: (0, 0)),
      ],
      out_specs=[pl.BlockSpec((_BLK, 128), lambda i: (i, 0))] * 6,
      out_shape=[jax.ShapeDtypeStruct((NPAD, 128), F32)] * 6,
  )(xp, W1, ws, wd)


def _comb_act(r_refs, den_ref, b_ref):
  d = den_ref[0] + den_ref[1]
  inv = 1.0 / (d + 1e-16)
  cols = []
  for h in range(HEADS):
    rr = r_refs[h // 2]
    blk = (rr[0] + rr[1])[:, (h % 2) * 64:(h % 2) * 64 + 64]
    cols.append(blk * inv[:, 16 * h:16 * h + 1])
  act = jnp.concatenate(cols, axis=1) + b_ref[...]
  return jnp.maximum(act, 0.0)


def _comb_lift_body(cw, r0, r1, r2, r3, den_ref, b_ref, w_ref,
                    ws_ref, wd_ref, *outs):
  act = _comb_act([r0, r1, r2, r3], den_ref, b_ref)
  h = jnp.dot(act, w_ref[...], preferred_element_type=F32)
  nchunks = cw * 0 + (F // 128 if cw == 128 else 1)
  hc = h
  if cw < 128:
    hc = jnp.concatenate([h, jnp.zeros((h.shape[0], 128 - cw), F32)], axis=1)
  for c in range(nchunks):
    outs[c][...] = hc[:, c * 128:c * 128 + 128] if cw == 128 else hc
  as_out = jnp.dot(h, ws_ref[...], preferred_element_type=F32)
  ad_out = jnp.dot(h, wd_ref[...], preferred_element_type=F32)
  outs[nchunks][...] = as_out
  outs[nchunks + 1][...] = ad_out


def _comb_lift(raws, den, b, W, ws, wd, fout):
  nchunks = 4 if fout == F else 1
  cw = fout // nchunks
  body = functools.partial(_comb_lift_body, cw)
  return pl.pallas_call(
      body,
      grid=(_GRID,),
      in_specs=[pl.BlockSpec((NC, _BLK, 128), lambda i: (0, i, 0))] * 4
      + [
          pl.BlockSpec((NC, _BLK, 128), lambda i: (0, i, 0)),
          pl.BlockSpec((1, F), lambda i: (0, 0)),
          pl.BlockSpec((F, fout), lambda i: (0, 0)),
          pl.BlockSpec((fout, 128), lambda i: (0, 0)),
          pl.BlockSpec((fout, 128), lambda i: (0, 0)),
      ],
      out_specs=[pl.BlockSpec((_BLK, 128), lambda i: (i, 0))] * (nchunks + 2),
      out_shape=[jax.ShapeDtypeStruct((NPAD, 128), F32)] * (nchunks + 2),
  )(*raws, den, b, W, ws, wd)


def _comb3_body(r_ref, den_ref, b_ref, out_ref):
  d = den_ref[0] + den_ref[1]
  inv = 1.0 / (d[:, 0:1] + 1e-16)
  out_ref[...] = (r_ref[0] + r_ref[1]) * inv + b_ref[...]


def _comb3(raw3, den3, b3p):
  return pl.pallas_call(
      _comb3_body,
      grid=(_GRID,),
      in_specs=[
          pl.BlockSpec((NC, _BLK, 128), lambda i: (0, i, 0)),
          pl.BlockSpec((NC, _BLK, 128), lambda i: (0, i, 0)),
          pl.BlockSpec((1, 128), lambda i: (0, 0)),
      ],
      out_specs=pl.BlockSpec((_BLK, 128), lambda i: (i, 0)),
      out_shape=jax.ShapeDtypeStruct((NPAD, 128), F32),
  )(raw3, den3, b3p)


def _mlp_body(l_ref, i_ref, w1a_ref, w1b_ref, b1_ref, w2_ref, b2_ref,
              w3_ref, b3_ref, out_ref):
  z = (jnp.dot(l_ref[...], w1a_ref[...], preferred_element_type=F32)
       + jnp.dot(i_ref[...], w1b_ref[...], preferred_element_type=F32)
       + b1_ref[...])
  z = jnp.maximum(z, 0.0)
  z = jnp.dot(z, w2_ref[...], preferred_element_type=F32) + b2_ref[...]
  z = jnp.maximum(z, 0.0)
  z = jnp.dot(z, w3_ref[...], preferred_element_type=F32) + b3_ref[...]
  out_ref[...] = 1.0 / (1.0 + jnp.exp(-z))


def _mlp(l_rows, i_rows, w1a, w1b, bm1, w2p, bm2p, w3p, bm3p):
  return pl.pallas_call(
      _mlp_body,
      grid=(1,),
      in_specs=[
          pl.BlockSpec((1024, 128), lambda i: (0, 0)),
          pl.BlockSpec((1024, 128), lambda i: (0, 0)),
          pl.BlockSpec((128, 64), lambda i: (0, 0)),
          pl.BlockSpec((128, 64), lambda i: (0, 0)),
          pl.BlockSpec((1, 64), lambda i: (0, 0)),
          pl.BlockSpec((64, 128), lambda i: (0, 0)),
          pl.BlockSpec((1, 128), lambda i: (0, 0)),
          pl.BlockSpec((128, 128), lambda i: (0, 0)),
          pl.BlockSpec((1, 128), lambda i: (0, 0)),
      ],
      out_specs=pl.BlockSpec((1024, 128), lambda i: (0, 0)),
      out_shape=jax.ShapeDtypeStruct((1024, 128), F32),
  )(l_rows, i_rows, w1a, w1b, bm1, w2p, bm2p, w3p, bm3p)


def _att_proj(a):
  Hh, C = a.shape
  if Hh == 1:
    return jnp.repeat(a.reshape(C, 1), 128, axis=1)
  M = jnp.zeros((Hh, C, Hh), F32)
  M = M.at[jnp.arange(Hh), :, jnp.arange(Hh)].set(a)
  M = M.reshape(Hh * C, Hh)
  return jnp.repeat(M, 16, axis=1)


def kernel(x, edge_index, liquor_idx, ingredient_idx,
           W1, a_src1, a_dst1, b1, W2, a_src2, a_dst2, b2,
           W3, a_src3, a_dst3, b3, Wm1, bm1, Wm2, bm2, Wm3, bm3):
  xp = jnp.zeros((NPAD, D), F32).at[:N].set(x)
  loop = jnp.arange(N, dtype=I32)
  padE = jnp.full((EPAD - E,), N, I32)
  src = jnp.concatenate([edge_index[0].astype(I32), loop, padE])
  dst = jnp.concatenate([edge_index[1].astype(I32), loop, padE])
  z128 = jnp.zeros((RPT, 128), F32)

  ws1, wd1 = _att_proj(a_src1), _att_proj(a_dst1)
  ws2, wd2 = _att_proj(a_src2), _att_proj(a_dst2)
  ws3, wd3 = _att_proj(a_src3), _att_proj(a_dst3)
  b1r = b1.reshape(1, F)
  b2r = b2.reshape(1, F)
  b3p = jnp.zeros((1, 128), F32).at[0, :64].set(b3)
  w1a = jnp.zeros((128, 64), F32).at[:64].set(Wm1[:64])
  w1b = jnp.zeros((128, 64), F32).at[:64].set(Wm1[64:])
  bm1r = bm1.reshape(1, 64)
  w2p = jnp.zeros((64, 128), F32).at[:, :32].set(Wm2)
  bm2p = jnp.zeros((1, 128), F32).at[0, :32].set(bm2)
  w3p = jnp.zeros((128, 128), F32).at[:32, 0:1].set(Wm3)
  bm3p = jnp.zeros((1, 128), F32).at[0, 0:1].set(bm3)

  h0, h1, h2, h3c, as1, ad1 = _lift1(xp, W1, ws1, wd1)
  exs1, den1 = _phase1(as1, ad1, src, dst, z128, 4)
  raw1 = _phase2((h0, h1, h2, h3c), exs1, src, dst, z128)

  g0, g1, g2, g3, as2, ad2 = _comb_lift(raw1, den1, b1r, W2, ws2, wd2, F)
  exs2, den2 = _phase1(as2, ad2, src, dst, z128, 4)
  raw2 = _phase2((g0, g1, g2, g3), exs2, src, dst, z128)

  h3pre, as3, ad3 = _comb_lift(raw2, den2, b2r, W3, ws3, wd3, 64)
  exs3, den3 = _phase1(as3, ad3, src, dst, z128, 1)
  raw3 = _phase2_chunk(h3pre, exs3[0], src, dst, z128)
  h3 = _comb3(raw3, den3, b3p)

  l_rows, i_rows = _pair_gather(h3, liquor_idx.astype(I32),
                                ingredient_idx.astype(I32))
  out = _mlp(l_rows, i_rows, w1a, w1b, bm1r, w2p, bm2p, w3p, bm3p)
  return out[:, 0]


# speedup vs baseline: 15.4500x; 1.1414x over previous
import functools

import jax
import jax.numpy as jnp
from jax import lax
from jax.experimental import pallas as pl
from jax.experimental.pallas import tpu as pltpu
from jax.experimental.pallas import tpu_sc as plsc

F32 = jnp.float32
I32 = jnp.int32

N = 10000
NPAD = 10240
D = 128
HID = 64
HEADS = 8
F = 512
E = 330000
B = 96
B1 = 32
NC, NS = 2, 16
NW = NC * NS
NB2 = 108
Q1 = NB2 * B
NB1 = Q1 // B1
EPAD = NW * Q1
NSC = 10112
RPT = NSC // NS

_mesh = plsc.VectorSubcoreMesh(
    core_axis_name="c", subcore_axis_name="s", num_cores=NC, num_subcores=NS)


def _p1_body(nex, *args):
  (as_hbm, ad_hbm, src_hbm, dst_hbm, z128_hbm) = args[:5]
  ex_outs = args[5:5 + nex]
  den_hbm = args[5 + nex]
  scr = args[6 + nex:]
  den_sh = scr[0]
  sidx = scr[1:5]
  didx = scr[5:9]
  asv = scr[9:11]
  adv = scr[11:13]
  denrow = scr[13]
  exc_vs = scr[14:14 + nex]
  gsem = scr[14 + nex:16 + nex]
  esem = scr[16 + nex]
  ssem = scr[17 + nex]

  cid = lax.axis_index("c")
  tid = lax.axis_index("s")
  wid = tid * NC + cid
  pltpu.sync_copy(z128_hbm.at[pl.ds(0, RPT)], den_sh.at[pl.ds(tid * RPT, RPT)])
  plsc.subcore_barrier()
  base = wid * Q1

  def load_and_gather(b, ib, buf):
    e0 = base + b * B1
    pltpu.sync_copy(src_hbm.at[pl.ds(e0, B1)], sidx[ib])
    pltpu.sync_copy(dst_hbm.at[pl.ds(e0, B1)], didx[ib])
    pltpu.async_copy(as_hbm.at[sidx[ib]], asv[buf], gsem[buf])
    pltpu.async_copy(ad_hbm.at[didx[ib]], adv[buf], gsem[buf])

  def drain_exc():
    for c in range(nex):
      pltpu.make_async_copy(
          exc_vs[c], ex_outs[c].at[pl.ds(0, B1)], esem).wait()

  def drain_den():
    pltpu.make_async_copy(denrow, den_sh.at[pl.ds(0, B1)], ssem).wait()

  load_and_gather(0, 0, 0)
  load_and_gather(1, 1, 1)

  def step(b, k):
    buf = k % 2
    as_v = asv[buf]
    ad_v = adv[buf]
    dr_v = denrow

    @pl.when(b >= 1)
    def _():
      drain_exc()
      drain_den()

    pltpu.make_async_copy(as_hbm.at[sidx[k]], as_v, gsem[buf]).wait()
    pltpu.make_async_copy(ad_hbm.at[didx[k]], ad_v, gsem[buf]).wait()
    e0 = base + b * B1

    def edge(i, _):
      for g in range(2 * nex):
        e16 = as_v[i, pl.ds(16 * g, 16)] + ad_v[i, pl.ds(16 * g, 16)]
        e16 = jnp.where(e16 >= 0.0, e16, e16 * 0.2)
        ex = jnp.exp(e16)
        dr_v[i, pl.ds(16 * g, 16)] = ex
        exc_vs[g // 2][i, pl.ds(16 * (g % 2), 16)] = ex
      return 0

    lax.fori_loop(0, B1, edge, 0, unroll=2)
    for c in range(nex):
      pltpu.async_copy(exc_vs[c], ex_outs[c].at[pl.ds(e0, B1)], esem)
    pltpu.async_copy(dr_v, den_sh.at[didx[k]], ssem, add=True)

    @pl.when(b + 2 < NB1)
    def _():
      load_and_gather(b + 2, (k + 2) % 4, buf)

  def loop4(it, _):
    b0 = 4 * it
    for k in range(4):
      step(b0 + k, k)
    return 0

  lax.fori_loop(0, NB1 // 4, loop4, 0)
  drain_exc()
  drain_den()
  plsc.subcore_barrier()
  pltpu.sync_copy(den_sh.at[pl.ds(tid * RPT, RPT)],
                  den_hbm.at[cid, pl.ds(tid * RPT, RPT)])


def _phase1(as_tab, ad_tab, src, dst, z128, nex):
  fn = pl.kernel(
      functools.partial(_p1_body, nex),
      out_type=(tuple(jax.ShapeDtypeStruct((EPAD, 32), F32)
                      for _ in range(nex))
                + (jax.ShapeDtypeStruct((NC, NPAD, 128), F32),)),
      mesh=_mesh,
      scratch_types=[
          pltpu.VMEM_SHARED((NSC, 128), F32),
      ] + [pltpu.VMEM((B1,), I32) for _ in range(8)]
      + [pltpu.VMEM((B1, 128), F32) for _ in range(5)]
      + [pltpu.VMEM((B1, 32), F32) for _ in range(nex)]
      + [pltpu.SemaphoreType.DMA for _ in range(4)],
  )
  outs = fn(as_tab, ad_tab, src, dst, z128)
  return outs[:nex], outs[nex]


def _p2_body(h_hbm, exc_hbm, src_hbm, dst_hbm, z128_hbm,
             raw_hbm,
             acc_sh, *scr):
  sidx = scr[0:4]
  didx = scr[4:8]
  exv = scr[8:10]
  gv = scr[10:12]
  gsem = scr[12:14]
  ssem = scr[14:16]
  sxsem = scr[16:18]
  dxsem = scr[18:20]
  exsem = scr[20:22]

  cid = lax.axis_index("c")
  tid = lax.axis_index("s")
  wid = tid * NC + cid
  pltpu.sync_copy(z128_hbm.at[pl.ds(0, RPT)], acc_sh.at[pl.ds(tid * RPT, RPT)])
  plsc.subcore_barrier()
  base = wid * Q1

  for b0 in range(2):
    e0 = base + b0 * B
    pltpu.sync_copy(src_hbm.at[pl.ds(e0, B)], sidx[b0])
    pltpu.sync_copy(dst_hbm.at[pl.ds(e0, B)], didx[b0])
    pltpu.sync_copy(exc_hbm.at[pl.ds(e0, B)], exv[b0])
    pltpu.async_copy(h_hbm.at[sidx[b0]], gv[b0], gsem[b0])

  def step(b, k):
    buf = k % 2
    g_v = gv[buf]
    ex_v = exv[buf]
    k2 = (k + 2) % 4
    e2 = base + (b + 2) * B

    @pl.when(b >= 2)
    def _():
      pltpu.make_async_copy(g_v, acc_sh.at[pl.ds(0, B)], ssem[buf]).wait()
      pltpu.make_async_copy(dst_hbm.at[pl.ds(0, B)], didx[k],
                            dxsem[buf]).wait()
      pltpu.make_async_copy(exc_hbm.at[pl.ds(0, B)], ex_v, exsem[buf]).wait()

    pltpu.make_async_copy(h_hbm.at[sidx[k]], g_v, gsem[buf]).wait()

    @pl.when(b + 2 < NB2)
    def _():
      pltpu.async_copy(src_hbm.at[pl.ds(e2, B)], sidx[k2], sxsem[buf])
      pltpu.async_copy(dst_hbm.at[pl.ds(e2, B)], didx[k2], dxsem[buf])

    def edge(i, _):
      s0 = ex_v[i, pl.ds(0, 16)]
      s1 = ex_v[i, pl.ds(16, 16)]
      for j in range(8):
        s = s0 if j < 4 else s1
        g_v[i, pl.ds(j * 16, 16)] = g_v[i, pl.ds(j * 16, 16)] * s
      return 0

    lax.fori_loop(0, B, edge, 0, unroll=2)
    pltpu.async_copy(g_v, acc_sh.at[didx[k]], ssem[buf], add=True)

    @pl.when(b + 2 < NB2)
    def _():
      pltpu.async_copy(exc_hbm.at[pl.ds(e2, B)], ex_v, exsem[buf])
      pltpu.make_async_copy(src_hbm.at[pl.ds(0, B)], sidx[k2],
                            sxsem[buf]).wait()
      pltpu.async_copy(h_hbm.at[sidx[k2]], g_v, gsem[buf])

  def loop4(it, _):
    b0 = 4 * it
    for k in range(4):
      step(b0 + k, k)
    return 0

  lax.fori_loop(0, NB2 // 4, loop4, 0)
  for buf in range(2):
    pltpu.make_async_copy(gv[buf], acc_sh.at[pl.ds(0, B)], ssem[buf]).wait()
  plsc.subcore_barrier()
  pltpu.sync_copy(acc_sh.at[pl.ds(tid * RPT, RPT)],
                  raw_hbm.at[cid, pl.ds(tid * RPT, RPT)])


def _phase2_chunk(h_chunk, exc, src, dst, z128):
  fn = pl.kernel(
      _p2_body,
      out_type=jax.ShapeDtypeStruct((NC, NPAD, 128), F32),
      mesh=_mesh,
      scratch_types=[
          pltpu.VMEM_SHARED((NSC, 128), F32),
      ] + [pltpu.VMEM((B,), I32) for _ in range(8)]
      + [pltpu.VMEM((B, 32), F32) for _ in range(2)]
      + [pltpu.VMEM((B, 128), F32) for _ in range(2)]
      + [pltpu.SemaphoreType.DMA for _ in range(10)],
  )
  return fn(h_chunk, exc, src, dst, z128)


def _phase2(h_chunks, exs, src, dst, z128):
  return [_phase2_chunk(h_chunks[c], exs[c], src, dst, z128)
          for c in range(4)]


def _pair_body(h3_hbm, liq_hbm, ing_hbm, l_out, i_out, idx_v, rows_v, sem):
  cid = lax.axis_index("c")
  tid = lax.axis_index("s")
  wid = tid * NC + cid
  base = wid * 32
  pltpu.sync_copy(liq_hbm.at[pl.ds(base, 32)], idx_v)
  pltpu.async_copy(h3_hbm.at[idx_v], rows_v, sem).wait()
  pltpu.sync_copy(rows_v, l_out.at[pl.ds(base, 32)])
  pltpu.sync_copy(ing_hbm.at[pl.ds(base, 32)], idx_v)
  pltpu.async_copy(h3_hbm.at[idx_v], rows_v, sem).wait()
  pltpu.sync_copy(rows_v, i_out.at[pl.ds(base, 32)])


def _pair_gather(h3, liq, ing):
  fn = pl.kernel(
      _pair_body,
      out_type=(jax.ShapeDtypeStruct((1024, 128), F32),
                jax.ShapeDtypeStruct((1024, 128), F32)),
      mesh=_mesh,
      scratch_types=[
          pltpu.VMEM((32,), I32),
          pltpu.VMEM((32, 128), F32),
          pltpu.SemaphoreType.DMA,
      ],
  )
  return fn(h3, liq, ing)


_BLK = 256
_GRID = NPAD // _BLK


def _lift1_body(x_ref, w_ref, ws_ref, wd_ref, h0, h1, h2, h3, as_ref, ad_ref):
  h = jnp.dot(x_ref[...], w_ref[...], preferred_element_type=F32)
  h0[...] = h[:, 0:128]
  h1[...] = h[:, 128:256]
  h2[...] = h[:, 256:384]
  h3[...] = h[:, 384:512]
  as_ref[...] = jnp.dot(h, ws_ref[...], preferred_element_type=F32)
  ad_ref[...] = jnp.dot(h, wd_ref[...], preferred_element_type=F32)


def _lift1(xp, W1, ws, wd):
  return pl.pallas_call(
      _lift1_body,
      grid=(_GRID,),
      in_specs=[
          pl.BlockSpec((_BLK, D), lambda i: (i, 0)),
          pl.BlockSpec((D, F), lambda i: (0, 0)),
          pl.BlockSpec((F, 128), lambda i: (0, 0)),
          pl.BlockSpec((F, 128), lambda i: (0, 0)),
      ],
      out_specs=[pl.BlockSpec((_BLK, 128), lambda i: (i, 0))] * 6,
      out_shape=[jax.ShapeDtypeStruct((NPAD, 128), F32)] * 6,
  )(xp, W1, ws, wd)


def _comb_act(r_refs, den_ref, b_ref):
  d = den_ref[0] + den_ref[1]
  inv = 1.0 / (d + 1e-16)
  cols = []
  for h in range(HEADS):
    rr = r_refs[h // 2]
    blk = (rr[0] + rr[1])[:, (h % 2) * 64:(h % 2) * 64 + 64]
    cols.append(blk * inv[:, 16 * h:16 * h + 1])
  act = jnp.concatenate(cols, axis=1) + b_ref[...]
  return jnp.maximum(act, 0.0)


def _comb_lift_body(cw, r0, r1, r2, r3, den_ref, b_ref, w_ref,
                    ws_ref, wd_ref, *outs):
  act = _comb_act([r0, r1, r2, r3], den_ref, b_ref)
  h = jnp.dot(act, w_ref[...], preferred_element_type=F32)
  nchunks = cw * 0 + (F // 128 if cw == 128 else 1)
  hc = h
  if cw < 128:
    hc = jnp.concatenate([h, jnp.zeros((h.shape[0], 128 - cw), F32)], axis=1)
  for c in range(nchunks):
    outs[c][...] = hc[:, c * 128:c * 128 + 128] if cw == 128 else hc
  as_out = jnp.dot(h, ws_ref[...], preferred_element_type=F32)
  ad_out = jnp.dot(h, wd_ref[...], preferred_element_type=F32)
  outs[nchunks][...] = as_out
  outs[nchunks + 1][...] = ad_out


def _comb_lift(raws, den, b, W, ws, wd, fout):
  nchunks = 4 if fout == F else 1
  cw = fout // nchunks
  body = functools.partial(_comb_lift_body, cw)
  return pl.pallas_call(
      body,
      grid=(_GRID,),
      in_specs=[pl.BlockSpec((NC, _BLK, 128), lambda i: (0, i, 0))] * 4
      + [
          pl.BlockSpec((NC, _BLK, 128), lambda i: (0, i, 0)),
          pl.BlockSpec((1, F), lambda i: (0, 0)),
          pl.BlockSpec((F, fout), lambda i: (0, 0)),
          pl.BlockSpec((fout, 128), lambda i: (0, 0)),
          pl.BlockSpec((fout, 128), lambda i: (0, 0)),
      ],
      out_specs=[pl.BlockSpec((_BLK, 128), lambda i: (i, 0))] * (nchunks + 2),
      out_shape=[jax.ShapeDtypeStruct((NPAD, 128), F32)] * (nchunks + 2),
  )(*raws, den, b, W, ws, wd)


def _comb3_body(r_ref, den_ref, b_ref, out_ref):
  d = den_ref[0] + den_ref[1]
  inv = 1.0 / (d[:, 0:1] + 1e-16)
  out_ref[...] = (r_ref[0] + r_ref[1]) * inv + b_ref[...]


def _comb3(raw3, den3, b3p):
  return pl.pallas_call(
      _comb3_body,
      grid=(_GRID,),
      in_specs=[
          pl.BlockSpec((NC, _BLK, 128), lambda i: (0, i, 0)),
          pl.BlockSpec((NC, _BLK, 128), lambda i: (0, i, 0)),
          pl.BlockSpec((1, 128), lambda i: (0, 0)),
      ],
      out_specs=pl.BlockSpec((_BLK, 128), lambda i: (i, 0)),
      out_shape=jax.ShapeDtypeStruct((NPAD, 128), F32),
  )(raw3, den3, b3p)


def _mlp_body(l_ref, i_ref, w1a_ref, w1b_ref, b1_ref, w2_ref, b2_ref,
              w3_ref, b3_ref, out_ref):
  z = (jnp.dot(l_ref[...], w1a_ref[...], preferred_element_type=F32)
       + jnp.dot(i_ref[...], w1b_ref[...], preferred_element_type=F32)
       + b1_ref[...])
  z = jnp.maximum(z, 0.0)
  z = jnp.dot(z, w2_ref[...], preferred_element_type=F32) + b2_ref[...]
  z = jnp.maximum(z, 0.0)
  z = jnp.dot(z, w3_ref[...], preferred_element_type=F32) + b3_ref[...]
  out_ref[...] = 1.0 / (1.0 + jnp.exp(-z))


def _mlp(l_rows, i_rows, w1a, w1b, bm1, w2p, bm2p, w3p, bm3p):
  return pl.pallas_call(
      _mlp_body,
      grid=(1,),
      in_specs=[
          pl.BlockSpec((1024, 128), lambda i: (0, 0)),
          pl.BlockSpec((1024, 128), lambda i: (0, 0)),
          pl.BlockSpec((128, 64), lambda i: (0, 0)),
          pl.BlockSpec((128, 64), lambda i: (0, 0)),
          pl.BlockSpec((1, 64), lambda i: (0, 0)),
          pl.BlockSpec((64, 128), lambda i: (0, 0)),
          pl.BlockSpec((1, 128), lambda i: (0, 0)),
          pl.BlockSpec((128, 128), lambda i: (0, 0)),
          pl.BlockSpec((1, 128), lambda i: (0, 0)),
      ],
      out_specs=pl.BlockSpec((1024, 128), lambda i: (0, 0)),
      out_shape=jax.ShapeDtypeStruct((1024, 128), F32),
  )(l_rows, i_rows, w1a, w1b, bm1, w2p, bm2p, w3p, bm3p)


def _att_proj(a):
  Hh, C = a.shape
  if Hh == 1:
    return jnp.repeat(a.reshape(C, 1), 128, axis=1)
  M = jnp.zeros((Hh, C, Hh), F32)
  M = M.at[jnp.arange(Hh), :, jnp.arange(Hh)].set(a)
  M = M.reshape(Hh * C, Hh)
  return jnp.repeat(M, 16, axis=1)


def kernel(x, edge_index, liquor_idx, ingredient_idx,
           W1, a_src1, a_dst1, b1, W2, a_src2, a_dst2, b2,
           W3, a_src3, a_dst3, b3, Wm1, bm1, Wm2, bm2, Wm3, bm3):
  xp = jnp.zeros((NPAD, D), F32).at[:N].set(x)
  loop = jnp.arange(N, dtype=I32)
  padE = jnp.full((EPAD - E,), N, I32)
  src = jnp.concatenate([edge_index[0].astype(I32), loop, padE])
  dst = jnp.concatenate([edge_index[1].astype(I32), loop, padE])
  z128 = jnp.zeros((RPT, 128), F32)

  ws1, wd1 = _att_proj(a_src1), _att_proj(a_dst1)
  ws2, wd2 = _att_proj(a_src2), _att_proj(a_dst2)
  ws3, wd3 = _att_proj(a_src3), _att_proj(a_dst3)
  b1r = b1.reshape(1, F)
  b2r = b2.reshape(1, F)
  b3p = jnp.zeros((1, 128), F32).at[0, :64].set(b3)
  w1a = jnp.zeros((128, 64), F32).at[:64].set(Wm1[:64])
  w1b = jnp.zeros((128, 64), F32).at[:64].set(Wm1[64:])
  bm1r = bm1.reshape(1, 64)
  w2p = jnp.zeros((64, 128), F32).at[:, :32].set(Wm2)
  bm2p = jnp.zeros((1, 128), F32).at[0, :32].set(bm2)
  w3p = jnp.zeros((128, 128), F32).at[:32, 0:1].set(Wm3)
  bm3p = jnp.zeros((1, 128), F32).at[0, 0:1].set(bm3)

  h0, h1, h2, h3c, as1, ad1 = _lift1(xp, W1, ws1, wd1)
  exs1, den1 = _phase1(as1, ad1, src, dst, z128, 4)
  raw1 = _phase2((h0, h1, h2, h3c), exs1, src, dst, z128)

  g0, g1, g2, g3, as2, ad2 = _comb_lift(raw1, den1, b1r, W2, ws2, wd2, F)
  exs2, den2 = _phase1(as2, ad2, src, dst, z128, 4)
  raw2 = _phase2((g0, g1, g2, g3), exs2, src, dst, z128)

  h3pre, as3, ad3 = _comb_lift(raw2, den2, b2r, W3, ws3, wd3, 64)
  exs3, den3 = _phase1(as3, ad3, src, dst, z128, 1)
  raw3 = _phase2_chunk(h3pre, exs3[0], src, dst, z128)
  h3 = _comb3(raw3, den3, b3p)

  l_rows, i_rows = _pair_gather(h3, liquor_idx.astype(I32),
                                ingredient_idx.astype(I32))
  out = _mlp(l_rows, i_rows, w1a, w1b, bm1r, w2p, bm2p, w3p, bm3p)
  return out[:, 0]


# speedup vs baseline: 16.1756x; 1.0470x over previous
import functools

import jax
import jax.numpy as jnp
from jax import lax
from jax.experimental import pallas as pl
from jax.experimental.pallas import tpu as pltpu
from jax.experimental.pallas import tpu_sc as plsc

F32 = jnp.float32
I32 = jnp.int32

N = 10000
NPAD = 10240
D = 128
HID = 64
HEADS = 8
F = 512
E = 330000
B = 96
B1 = 32
NC, NS = 2, 16
NW = NC * NS
NB2 = 108
Q1 = NB2 * B
NB1 = Q1 // B1
EPAD = NW * Q1
NSC = 10112
RPT = NSC // NS

_mesh = plsc.VectorSubcoreMesh(
    core_axis_name="c", subcore_axis_name="s", num_cores=NC, num_subcores=NS)


def _p1_body(nex, *args):
  (as_hbm, ad_hbm, src_hbm, dst_hbm, z128_hbm) = args[:5]
  ex_outs = args[5:5 + nex]
  den_hbm = args[5 + nex]
  scr = args[6 + nex:]
  den_sh = scr[0]
  sidx = scr[1:5]
  didx = scr[5:9]
  asv = scr[9:11]
  adv = scr[11:13]
  denrow = scr[13]
  exc_vs = scr[14:14 + nex]
  gsem = scr[14 + nex:16 + nex]
  esem = scr[16 + nex]
  ssem = scr[17 + nex]
  sxsem = scr[18 + nex]
  dxsem = scr[19 + nex]

  cid = lax.axis_index("c")
  tid = lax.axis_index("s")
  wid = tid * NC + cid
  pltpu.sync_copy(z128_hbm.at[pl.ds(0, RPT)], den_sh.at[pl.ds(tid * RPT, RPT)])
  plsc.subcore_barrier()
  base = wid * Q1

  def load_and_gather(b, ib, buf):
    e0 = base + b * B1
    pltpu.sync_copy(src_hbm.at[pl.ds(e0, B1)], sidx[ib])
    pltpu.sync_copy(dst_hbm.at[pl.ds(e0, B1)], didx[ib])
    pltpu.async_copy(as_hbm.at[sidx[ib]], asv[buf], gsem[buf])
    pltpu.async_copy(ad_hbm.at[didx[ib]], adv[buf], gsem[buf])

  def drain_exc():
    for c in range(nex):
      pltpu.make_async_copy(
          exc_vs[c], ex_outs[c].at[pl.ds(0, B1)], esem).wait()

  def drain_den():
    pltpu.make_async_copy(denrow, den_sh.at[pl.ds(0, B1)], ssem).wait()

  load_and_gather(0, 0, 0)
  load_and_gather(1, 1, 1)

  def step(b, k):
    buf = k % 2
    as_v = asv[buf]
    ad_v = adv[buf]
    dr_v = denrow

    @pl.when(b >= 1)
    def _():
      drain_exc()
      drain_den()

    pltpu.make_async_copy(as_hbm.at[sidx[k]], as_v, gsem[buf]).wait()
    pltpu.make_async_copy(ad_hbm.at[didx[k]], ad_v, gsem[buf]).wait()
    e0 = base + b * B1
    k2 = (k + 2) % 4
    e2 = base + (b + 2) * B1

    @pl.when(b + 2 < NB1)
    def _():
      pltpu.async_copy(src_hbm.at[pl.ds(e2, B1)], sidx[k2], sxsem)
      pltpu.async_copy(dst_hbm.at[pl.ds(e2, B1)], didx[k2], dxsem)

    def edge(i, _):
      for g in range(2 * nex):
        e16 = as_v[i, pl.ds(16 * g, 16)] + ad_v[i, pl.ds(16 * g, 16)]
        e16 = jnp.where(e16 >= 0.0, e16, e16 * 0.2)
        ex = jnp.exp(e16)
        dr_v[i, pl.ds(16 * g, 16)] = ex
        exc_vs[g // 2][i, pl.ds(16 * (g % 2), 16)] = ex
      return 0

    lax.fori_loop(0, B1, edge, 0, unroll=2)
    for c in range(nex):
      pltpu.async_copy(exc_vs[c], ex_outs[c].at[pl.ds(e0, B1)], esem)
    pltpu.async_copy(dr_v, den_sh.at[didx[k]], ssem, add=True)

    @pl.when(b + 2 < NB1)
    def _():
      pltpu.make_async_copy(src_hbm.at[pl.ds(0, B1)], sidx[k2], sxsem).wait()
      pltpu.make_async_copy(dst_hbm.at[pl.ds(0, B1)], didx[k2], dxsem).wait()
      pltpu.async_copy(as_hbm.at[sidx[k2]], as_v, gsem[buf])
      pltpu.async_copy(ad_hbm.at[didx[k2]], ad_v, gsem[buf])

  def loop4(it, _):
    b0 = 4 * it
    for k in range(4):
      step(b0 + k, k)
    return 0

  lax.fori_loop(0, NB1 // 4, loop4, 0)
  drain_exc()
  drain_den()
  plsc.subcore_barrier()
  pltpu.sync_copy(den_sh.at[pl.ds(tid * RPT, RPT)],
                  den_hbm.at[cid, pl.ds(tid * RPT, RPT)])


def _phase1(as_tab, ad_tab, src, dst, z128, nex):
  fn = pl.kernel(
      functools.partial(_p1_body, nex),
      out_type=(tuple(jax.ShapeDtypeStruct((EPAD, 32), F32)
                      for _ in range(nex))
                + (jax.ShapeDtypeStruct((NC, NPAD, 128), F32),)),
      mesh=_mesh,
      scratch_types=[
          pltpu.VMEM_SHARED((NSC, 128), F32),
      ] + [pltpu.VMEM((B1,), I32) for _ in range(8)]
      + [pltpu.VMEM((B1, 128), F32) for _ in range(5)]
      + [pltpu.VMEM((B1, 32), F32) for _ in range(nex)]
      + [pltpu.SemaphoreType.DMA for _ in range(6)],
  )
  outs = fn(as_tab, ad_tab, src, dst, z128)
  return outs[:nex], outs[nex]


def _p2_body(h_hbm, exc_hbm, src_hbm, dst_hbm, z128_hbm,
             raw_hbm,
             acc_sh, *scr):
  sidx = scr[0:4]
  didx = scr[4:8]
  exv = scr[8:10]
  gv = scr[10:12]
  gsem = scr[12:14]
  ssem = scr[14:16]
  sxsem = scr[16:18]
  dxsem = scr[18:20]
  exsem = scr[20:22]

  cid = lax.axis_index("c")
  tid = lax.axis_index("s")
  wid = tid * NC + cid
  pltpu.sync_copy(z128_hbm.at[pl.ds(0, RPT)], acc_sh.at[pl.ds(tid * RPT, RPT)])
  plsc.subcore_barrier()
  base = wid * Q1

  for b0 in range(2):
    e0 = base + b0 * B
    pltpu.sync_copy(src_hbm.at[pl.ds(e0, B)], sidx[b0])
    pltpu.sync_copy(dst_hbm.at[pl.ds(e0, B)], didx[b0])
    pltpu.sync_copy(exc_hbm.at[pl.ds(e0, B)], exv[b0])
    pltpu.async_copy(h_hbm.at[sidx[b0]], gv[b0], gsem[b0])

  def step(b, k):
    buf = k % 2
    g_v = gv[buf]
    ex_v = exv[buf]
    k2 = (k + 2) % 4
    e2 = base + (b + 2) * B

    @pl.when(b >= 2)
    def _():
      pltpu.make_async_copy(g_v, acc_sh.at[pl.ds(0, B)], ssem[buf]).wait()
      pltpu.make_async_copy(dst_hbm.at[pl.ds(0, B)], didx[k],
                            dxsem[buf]).wait()
      pltpu.make_async_copy(exc_hbm.at[pl.ds(0, B)], ex_v, exsem[buf]).wait()

    pltpu.make_async_copy(h_hbm.at[sidx[k]], g_v, gsem[buf]).wait()

    @pl.when(b + 2 < NB2)
    def _():
      pltpu.async_copy(src_hbm.at[pl.ds(e2, B)], sidx[k2], sxsem[buf])
      pltpu.async_copy(dst_hbm.at[pl.ds(e2, B)], didx[k2], dxsem[buf])

    def edge(i, _):
      s0 = ex_v[i, pl.ds(0, 16)]
      s1 = ex_v[i, pl.ds(16, 16)]
      for j in range(8):
        s = s0 if j < 4 else s1
        g_v[i, pl.ds(j * 16, 16)] = g_v[i, pl.ds(j * 16, 16)] * s
      return 0

    lax.fori_loop(0, B, edge, 0, unroll=2)
    pltpu.async_copy(g_v, acc_sh.at[didx[k]], ssem[buf], add=True)

    @pl.when(b + 2 < NB2)
    def _():
      pltpu.async_copy(exc_hbm.at[pl.ds(e2, B)], ex_v, exsem[buf])
      pltpu.make_async_copy(src_hbm.at[pl.ds(0, B)], sidx[k2],
                            sxsem[buf]).wait()
      pltpu.async_copy(h_hbm.at[sidx[k2]], g_v, gsem[buf])

  def loop4(it, _):
    b0 = 4 * it
    for k in range(4):
      step(b0 + k, k)
    return 0

  lax.fori_loop(0, NB2 // 4, loop4, 0)
  for buf in range(2):
    pltpu.make_async_copy(gv[buf], acc_sh.at[pl.ds(0, B)], ssem[buf]).wait()
  plsc.subcore_barrier()
  pltpu.sync_copy(acc_sh.at[pl.ds(tid * RPT, RPT)],
                  raw_hbm.at[cid, pl.ds(tid * RPT, RPT)])


def _phase2_chunk(h_chunk, exc, src, dst, z128):
  fn = pl.kernel(
      _p2_body,
      out_type=jax.ShapeDtypeStruct((NC, NPAD, 128), F32),
      mesh=_mesh,
      scratch_types=[
          pltpu.VMEM_SHARED((NSC, 128), F32),
      ] + [pltpu.VMEM((B,), I32) for _ in range(8)]
      + [pltpu.VMEM((B, 32), F32) for _ in range(2)]
      + [pltpu.VMEM((B, 128), F32) for _ in range(2)]
      + [pltpu.SemaphoreType.DMA for _ in range(10)],
  )
  return fn(h_chunk, exc, src, dst, z128)


def _phase2(h_chunks, exs, src, dst, z128):
  return [_phase2_chunk(h_chunks[c], exs[c], src, dst, z128)
          for c in range(4)]


def _pair_body(h3_hbm, liq_hbm, ing_hbm, l_out, i_out, idx_v, rows_v, sem):
  cid = lax.axis_index("c")
  tid = lax.axis_index("s")
  wid = tid * NC + cid
  base = wid * 32
  pltpu.sync_copy(liq_hbm.at[pl.ds(base, 32)], idx_v)
  pltpu.async_copy(h3_hbm.at[idx_v], rows_v, sem).wait()
  pltpu.sync_copy(rows_v, l_out.at[pl.ds(base, 32)])
  pltpu.sync_copy(ing_hbm.at[pl.ds(base, 32)], idx_v)
  pltpu.async_copy(h3_hbm.at[idx_v], rows_v, sem).wait()
  pltpu.sync_copy(rows_v, i_out.at[pl.ds(base, 32)])


def _pair_gather(h3, liq, ing):
  fn = pl.kernel(
      _pair_body,
      out_type=(jax.ShapeDtypeStruct((1024, 128), F32),
                jax.ShapeDtypeStruct((1024, 128), F32)),
      mesh=_mesh,
      scratch_types=[
          pltpu.VMEM((32,), I32),
          pltpu.VMEM((32, 128), F32),
          pltpu.SemaphoreType.DMA,
      ],
  )
  return fn(h3, liq, ing)


_BLK = 256
_GRID = NPAD // _BLK


def _lift1_body(x_ref, w_ref, ws_ref, wd_ref, h0, h1, h2, h3, as_ref, ad_ref):
  h = jnp.dot(x_ref[...], w_ref[...], preferred_element_type=F32)
  h0[...] = h[:, 0:128]
  h1[...] = h[:, 128:256]
  h2[...] = h[:, 256:384]
  h3[...] = h[:, 384:512]
  as_ref[...] = jnp.dot(h, ws_ref[...], preferred_element_type=F32)
  ad_ref[...] = jnp.dot(h, wd_ref[...], preferred_element_type=F32)


def _lift1(xp, W1, ws, wd):
  return pl.pallas_call(
      _lift1_body,
      grid=(_GRID,),
      in_specs=[
          pl.BlockSpec((_BLK, D), lambda i: (i, 0)),
          pl.BlockSpec((D, F), lambda i: (0, 0)),
          pl.BlockSpec((F, 128), lambda i: (0, 0)),
          pl.BlockSpec((F, 128), lambda i: (0, 0)),
      ],
      out_specs=[pl.BlockSpec((_BLK, 128), lambda i: (i, 0))] * 6,
      out_shape=[jax.ShapeDtypeStruct((NPAD, 128), F32)] * 6,
  )(xp, W1, ws, wd)


def _comb_act(r_refs, den_ref, b_ref):
  d = den_ref[0] + den_ref[1]
  inv = 1.0 / (d + 1e-16)
  cols = []
  for h in range(HEADS):
    rr = r_refs[h // 2]
    blk = (rr[0] + rr[1])[:, (h % 2) * 64:(h % 2) * 64 + 64]
    cols.append(blk * inv[:, 16 * h:16 * h + 1])
  act = jnp.concatenate(cols, axis=1) + b_ref[...]
  return jnp.maximum(act, 0.0)


def _comb_lift_body(cw, r0, r1, r2, r3, den_ref, b_ref, w_ref,
                    ws_ref, wd_ref, *outs):
  act = _comb_act([r0, r1, r2, r3], den_ref, b_ref)
  h = jnp.dot(act, w_ref[...], preferred_element_type=F32)
  nchunks = cw * 0 + (F // 128 if cw == 128 else 1)
  hc = h
  if cw < 128:
    hc = jnp.concatenate([h, jnp.zeros((h.shape[0], 128 - cw), F32)], axis=1)
  for c in range(nchunks):
    outs[c][...] = hc[:, c * 128:c * 128 + 128] if cw == 128 else hc
  as_out = jnp.dot(h, ws_ref[...], preferred_element_type=F32)
  ad_out = jnp.dot(h, wd_ref[...], preferred_element_type=F32)
  outs[nchunks][...] = as_out
  outs[nchunks + 1][...] = ad_out


def _comb_lift(raws, den, b, W, ws, wd, fout):
  nchunks = 4 if fout == F else 1
  cw = fout // nchunks
  body = functools.partial(_comb_lift_body, cw)
  return pl.pallas_call(
      body,
      grid=(_GRID,),
      in_specs=[pl.BlockSpec((NC, _BLK, 128), lambda i: (0, i, 0))] * 4
      + [
          pl.BlockSpec((NC, _BLK, 128), lambda i: (0, i, 0)),
          pl.BlockSpec((1, F), lambda i: (0, 0)),
          pl.BlockSpec((F, fout), lambda i: (0, 0)),
          pl.BlockSpec((fout, 128), lambda i: (0, 0)),
          pl.BlockSpec((fout, 128), lambda i: (0, 0)),
      ],
      out_specs=[pl.BlockSpec((_BLK, 128), lambda i: (i, 0))] * (nchunks + 2),
      out_shape=[jax.ShapeDtypeStruct((NPAD, 128), F32)] * (nchunks + 2),
  )(*raws, den, b, W, ws, wd)


def _comb3_body(r_ref, den_ref, b_ref, out_ref):
  d = den_ref[0] + den_ref[1]
  inv = 1.0 / (d[:, 0:1] + 1e-16)
  out_ref[...] = (r_ref[0] + r_ref[1]) * inv + b_ref[...]


def _comb3(raw3, den3, b3p):
  return pl.pallas_call(
      _comb3_body,
      grid=(_GRID,),
      in_specs=[
          pl.BlockSpec((NC, _BLK, 128), lambda i: (0, i, 0)),
          pl.BlockSpec((NC, _BLK, 128), lambda i: (0, i, 0)),
          pl.BlockSpec((1, 128), lambda i: (0, 0)),
      ],
      out_specs=pl.BlockSpec((_BLK, 128), lambda i: (i, 0)),
      out_shape=jax.ShapeDtypeStruct((NPAD, 128), F32),
  )(raw3, den3, b3p)


def _mlp_body(l_ref, i_ref, w1a_ref, w1b_ref, b1_ref, w2_ref, b2_ref,
              w3_ref, b3_ref, out_ref):
  z = (jnp.dot(l_ref[...], w1a_ref[...], preferred_element_type=F32)
       + jnp.dot(i_ref[...], w1b_ref[...], preferred_element_type=F32)
       + b1_ref[...])
  z = jnp.maximum(z, 0.0)
  z = jnp.dot(z, w2_ref[...], preferred_element_type=F32) + b2_ref[...]
  z = jnp.maximum(z, 0.0)
  z = jnp.dot(z, w3_ref[...], preferred_element_type=F32) + b3_ref[...]
  out_ref[...] = 1.0 / (1.0 + jnp.exp(-z))


def _mlp(l_rows, i_rows, w1a, w1b, bm1, w2p, bm2p, w3p, bm3p):
  return pl.pallas_call(
      _mlp_body,
      grid=(1,),
      in_specs=[
          pl.BlockSpec((1024, 128), lambda i: (0, 0)),
          pl.BlockSpec((1024, 128), lambda i: (0, 0)),
          pl.BlockSpec((128, 64), lambda i: (0, 0)),
          pl.BlockSpec((128, 64), lambda i: (0, 0)),
          pl.BlockSpec((1, 64), lambda i: (0, 0)),
          pl.BlockSpec((64, 128), lambda i: (0, 0)),
          pl.BlockSpec((1, 128), lambda i: (0, 0)),
          pl.BlockSpec((128, 128), lambda i: (0, 0)),
          pl.BlockSpec((1, 128), lambda i: (0, 0)),
      ],
      out_specs=pl.BlockSpec((1024, 128), lambda i: (0, 0)),
      out_shape=jax.ShapeDtypeStruct((1024, 128), F32),
  )(l_rows, i_rows, w1a, w1b, bm1, w2p, bm2p, w3p, bm3p)


def _att_proj(a):
  Hh, C = a.shape
  if Hh == 1:
    return jnp.repeat(a.reshape(C, 1), 128, axis=1)
  M = jnp.zeros((Hh, C, Hh), F32)
  M = M.at[jnp.arange(Hh), :, jnp.arange(Hh)].set(a)
  M = M.reshape(Hh * C, Hh)
  return jnp.repeat(M, 16, axis=1)


def kernel(x, edge_index, liquor_idx, ingredient_idx,
           W1, a_src1, a_dst1, b1, W2, a_src2, a_dst2, b2,
           W3, a_src3, a_dst3, b3, Wm1, bm1, Wm2, bm2, Wm3, bm3):
  xp = jnp.zeros((NPAD, D), F32).at[:N].set(x)
  loop = jnp.arange(N, dtype=I32)
  padE = jnp.full((EPAD - E,), N, I32)
  src = jnp.concatenate([edge_index[0].astype(I32), loop, padE])
  dst = jnp.concatenate([edge_index[1].astype(I32), loop, padE])
  z128 = jnp.zeros((RPT, 128), F32)

  ws1, wd1 = _att_proj(a_src1), _att_proj(a_dst1)
  ws2, wd2 = _att_proj(a_src2), _att_proj(a_dst2)
  ws3, wd3 = _att_proj(a_src3), _att_proj(a_dst3)
  b1r = b1.reshape(1, F)
  b2r = b2.reshape(1, F)
  b3p = jnp.zeros((1, 128), F32).at[0, :64].set(b3)
  w1a = jnp.zeros((128, 64), F32).at[:64].set(Wm1[:64])
  w1b = jnp.zeros((128, 64), F32).at[:64].set(Wm1[64:])
  bm1r = bm1.reshape(1, 64)
  w2p = jnp.zeros((64, 128), F32).at[:, :32].set(Wm2)
  bm2p = jnp.zeros((1, 128), F32).at[0, :32].set(bm2)
  w3p = jnp.zeros((128, 128), F32).at[:32, 0:1].set(Wm3)
  bm3p = jnp.zeros((1, 128), F32).at[0, 0:1].set(bm3)

  h0, h1, h2, h3c, as1, ad1 = _lift1(xp, W1, ws1, wd1)
  exs1, den1 = _phase1(as1, ad1, src, dst, z128, 4)
  raw1 = _phase2((h0, h1, h2, h3c), exs1, src, dst, z128)

  g0, g1, g2, g3, as2, ad2 = _comb_lift(raw1, den1, b1r, W2, ws2, wd2, F)
  exs2, den2 = _phase1(as2, ad2, src, dst, z128, 4)
  raw2 = _phase2((g0, g1, g2, g3), exs2, src, dst, z128)

  h3pre, as3, ad3 = _comb_lift(raw2, den2, b2r, W3, ws3, wd3, 64)
  exs3, den3 = _phase1(as3, ad3, src, dst, z128, 1)
  raw3 = _phase2_chunk(h3pre, exs3[0], src, dst, z128)
  h3 = _comb3(raw3, den3, b3p)

  l_rows, i_rows = _pair_gather(h3, liquor_idx.astype(I32),
                                ingredient_idx.astype(I32))
  out = _mlp(l_rows, i_rows, w1a, w1b, bm1r, w2p, bm2p, w3p, bm3p)
  return out[:, 0]
